# Initial kernel scaffold; baseline (speedup 1.0000x reference)
import jax
import jax.numpy as jnp
from jax.experimental import pallas as pl


def kernel(x, edge_index, edges1, edges2, edges3, pool1, pool2, pool3, pool4, W1, b1, W2, b2, W3, b3, W4, b4, fcW, fcb):
    raise NotImplementedError("write your pallas kernel here")


# speedup vs baseline: 5.2593x; 5.2593x over previous
import functools

import jax
import jax.numpy as jnp
from jax import lax
from jax.experimental import pallas as pl
from jax.experimental.pallas import tpu as pltpu
from jax.experimental.pallas import tpu_sc as plsc

F32 = jnp.float32
I32 = jnp.int32
KCH = 6

LVL = [
    dict(N=5632, Npad=5632, E=33792, CH=1024, Fi=3,  Fo=16, S=2112, R=352,
         Nn=1408, Nnp=1408, leaky=True),
    dict(N=1408, Npad=1536, E=8448,  CH=1056, Fi=16, Fo=16, S=528,  R=96,
         Nn=352, Nnp=352, leaky=False),
    dict(N=352,  Npad=512,  E=2112,  CH=704,  Fi=16, Fo=16, S=136,  R=32,
         Nn=88, Nnp=96, leaky=False),
    dict(N=88,   Npad=256,  E=528,   CH=528,  Fi=16, Fo=32, S=40,   R=16,
         Nn=22, Nnp=32, leaky=False),
]

NMAX = 5632
EMAX = 33792
CHMAX = 1056


def _splat(v, dt=I32):
  return jnp.full((16,), v, dt)


def _rsqrt_q(d):
  i = plsc.bitcast(d, I32)
  y = plsc.bitcast(0x5F3759DF - lax.shift_right_arithmetic(i, 1), F32)
  for _ in range(3):
    y = y * (1.5 - 0.5 * d * y * y)
  return y


def _zero(ref, n):
  def body(i, c):
    ref[pl.ds(i * 16, 16)] = jnp.zeros((16,), F32)
    return c
  lax.fori_loop(0, n // 16, body, 0)


def _sc_body(xT, s0, d0, s1, d1, s2, d2, s3, d3,
             p1, p2, p3, p4, W1, b1, W2, b2, W3, b3, W4, b4, fcW, fcb,
             out,
             A, B, C, O0, O1, TMPC, DIS, ES, ED, EN, PIDX,
             WB1, WB2, WB3, WB4, BB1, BB2, BB3, BB4, FCWv, FCBv, V16, HC,
             SH_H, SH_A, SH_P, SH_DIS, SH_EN, SH_RED,
             SH_ES0, SH_ED0, SH_ES1, SH_ED1, SH_ES2, SH_ED2, SH_ES3, SH_ED3):
  wid = lax.axis_index("s")
  iota = lax.iota(I32, 16)
  ones16 = jnp.ones((16,), F32)

  pltpu.sync_copy(W1, WB1)
  pltpu.sync_copy(W2, WB2)
  pltpu.sync_copy(W3, WB3)
  pltpu.sync_copy(W4, WB4)
  pltpu.sync_copy(b1, BB1)
  pltpu.sync_copy(b2, BB2)
  pltpu.sync_copy(b3, BB3)
  pltpu.sync_copy(b4, BB4)
  pltpu.sync_copy(fcW, FCWv)
  pltpu.sync_copy(fcb, FCBv)

  def matmul(k, WB, Fi, Fo, Npad):
    def jloop(j, c):
      pltpu.sync_copy(SH_A.at[pl.ds(pl.multiple_of(j * NMAX, 8), Npad)], TMPC.at[pl.ds(0, Npad)])
      jbase = jnp.full((16,), k * Fi * Fo, I32) + j * Fo
      w0 = plsc.load_gather(WB, [jbase + wid])
      if Fo > 16:
        w1 = plsc.load_gather(WB, [jbase + wid + 16])
      def vloop(i, c2):
        t = TMPC[pl.ds(i * 16, 16)]
        O0[pl.ds(i * 16, 16)] += t * w0
        if Fo > 16:
          O1[pl.ds(i * 16, 16)] += t * w1
        return c2
      lax.fori_loop(0, Npad // 16, vloop, 0)
      return c
    lax.fori_loop(0, Fi, jloop, 0)

  def prop(src_ref, dst_ref, E, CH, Npad, SH_ES, SH_ED):
    _zero(dst_ref, Npad)
    def chunk(ci, c):
      base = pl.multiple_of(ci * CH, 8)
      pltpu.sync_copy(SH_ES.at[pl.ds(base, CH)], ES.at[pl.ds(0, CH)])
      pltpu.sync_copy(SH_ED.at[pl.ds(base, CH)], ED.at[pl.ds(0, CH)])
      pltpu.sync_copy(SH_EN.at[pl.ds(base, CH)], EN.at[pl.ds(0, CH)])
      def vloop(j, c2):
        s = ES[pl.ds(j * 16, 16)]
        d = ED[pl.ds(j * 16, 16)]
        nv = EN[pl.ds(j * 16, 16)]
        g = plsc.load_gather(src_ref, [s])
        plsc.addupdate_scatter(dst_ref, [d], g * nv, mask=None)
        return c2
      lax.fori_loop(0, CH // 16, vloop, 0)
      return c
    lax.fori_loop(0, E // CH, chunk, 0)

  def level(l, e_src, e_dst, pool, WB, BB, SH_ES, SH_ED):
    lv = LVL[l]
    N, Npad, E, CH = lv["N"], lv["Npad"], lv["E"], lv["CH"]
    Fi, Fo, S, R = lv["Fi"], lv["Fo"], lv["S"], lv["R"]
    Nn, Nnp, leaky = lv["Nn"], lv["Nnp"], lv["leaky"]
    nch_sl = -(-S // CH)

    @pl.when(wid == 0)
    def _stage():
      pltpu.sync_copy(e_src, SH_ES)
      pltpu.sync_copy(e_dst, SH_ED)
    plsc.subcore_barrier()

    _zero(C, Npad)
    lo = wid * S
    hi = jnp.minimum(lo + S, E)

    def degchunk(ci, c):
      base = lo + ci * CH
      base2 = pl.multiple_of(jnp.maximum(jnp.minimum(base, E - CH), 0), 8)
      pltpu.sync_copy(SH_ED.at[pl.ds(base2, CH)], ED.at[pl.ds(0, CH)])
      def vloop(j, c2):
        pos = base2 + j * 16 + iota
        m = (pos >= base) & (pos < hi)
        dd = ED[pl.ds(j * 16, 16)]
        plsc.addupdate_scatter(C, [dd], jnp.where(m, 1.0, 0.0).astype(F32))
        return c2
      lax.fori_loop(0, CH // 16, vloop, 0)
      return c
    lax.fori_loop(0, nch_sl, degchunk, 0)
    pltpu.sync_copy(C.at[pl.ds(0, Npad)], SH_P.at[pl.ds(pl.multiple_of(wid * NMAX, 8), Npad)])
    plsc.subcore_barrier()

    rbase = pl.multiple_of(wid * R, 8)
    _zero(A, R)
    def pacc(p, c):
      pltpu.sync_copy(SH_P.at[pl.ds(pl.multiple_of(p * NMAX + rbase, 8), R)], TMPC.at[pl.ds(0, R)])
      def vloop(j, c2):
        A[pl.ds(j * 16, 16)] += TMPC[pl.ds(j * 16, 16)]
        return c2
      lax.fori_loop(0, R // 16, vloop, 0)
      return c
    lax.fori_loop(0, 16, pacc, 0)
    def vdis(j, c):
      d = A[pl.ds(j * 16, 16)]
      B[pl.ds(j * 16, 16)] = jnp.where(d > 0.0, _rsqrt_q(d), 0.0)
      return c
    lax.fori_loop(0, R // 16, vdis, 0)
    pltpu.sync_copy(B.at[pl.ds(0, R)], SH_DIS.at[pl.ds(rbase, R)])
    plsc.subcore_barrier()

    pltpu.sync_copy(SH_DIS.at[pl.ds(0, Npad)], DIS.at[pl.ds(0, Npad)])
    def nchunk(ci, c):
      base = lo + ci * CH
      base2 = pl.multiple_of(jnp.maximum(jnp.minimum(base, E - CH), 0), 8)
      pltpu.sync_copy(SH_ES.at[pl.ds(base2, CH)], ES.at[pl.ds(0, CH)])
      pltpu.sync_copy(SH_ED.at[pl.ds(base2, CH)], ED.at[pl.ds(0, CH)])
      def vloop(j, c2):
        s = ES[pl.ds(j * 16, 16)]
        d = ED[pl.ds(j * 16, 16)]
        g = plsc.load_gather(DIS, [s]) * plsc.load_gather(DIS, [d])
        EN[pl.ds(j * 16, 16)] = -g
        return c2
      lax.fori_loop(0, CH // 16, vloop, 0)
      pltpu.sync_copy(EN.at[pl.ds(0, CH)], SH_EN.at[pl.ds(base2, CH)])
      return c
    lax.fori_loop(0, nch_sl, nchunk, 0)
    plsc.subcore_barrier()

    bufs = [A, B, C]

    def init_col(a_ref):
      if l == 0:
        pltpu.sync_copy(xT.at[pl.ds(pl.multiple_of(wid * NMAX, 8), Npad)], a_ref.at[pl.ds(0, Npad)])
      else:
        pltpu.sync_copy(SH_H.at[pl.ds(pl.multiple_of(wid * NMAX, 8), Npad)], a_ref.at[pl.ds(0, Npad)])
      pltpu.sync_copy(a_ref.at[pl.ds(0, Npad)], SH_A.at[pl.ds(pl.multiple_of(wid * NMAX, 8), Npad)])

    pl.when(wid < Fi)(functools.partial(init_col, bufs[0]))
    _zero(O0, Npad)
    if Fo > 16:
      _zero(O1, Npad)
    plsc.subcore_barrier()
    matmul(0, WB, Fi, Fo, Npad)

    for k in range(1, KCH):
      a, b, c = bufs
      if k == 1:
        def step1(a=a, c=c):
          prop(a, c, E, CH, Npad, SH_ES, SH_ED)
        pl.when(wid < Fi)(step1)
        bufs = [a, c, b]
      else:
        def stepk(a=a, b=b, c=c):
          prop(b, c, E, CH, Npad, SH_ES, SH_ED)
          def tr(j, cc):
            C2 = c[pl.ds(j * 16, 16)]
            c[pl.ds(j * 16, 16)] = 2.0 * C2 - a[pl.ds(j * 16, 16)]
            return cc
          lax.fori_loop(0, Npad // 16, tr, 0)
        pl.when(wid < Fi)(stepk)
        bufs = [b, c, a]
      plsc.subcore_barrier()
      txk = bufs[1]
      def publish(txk=txk):
        pltpu.sync_copy(txk.at[pl.ds(0, Npad)], SH_A.at[pl.ds(pl.multiple_of(wid * NMAX, 8), Npad)])
      pl.when(wid < Fi)(publish)
      plsc.subcore_barrier()
      matmul(k, WB, Fi, Fo, Npad)

    def bias_act_pool(o_ref, col_off, row):
      bv = plsc.load_gather(BB, [_splat(col_off) + wid])
      def vact(i, c):
        v = o_ref[pl.ds(i * 16, 16)] + bv
        if leaky:
          v = jnp.maximum(v, 0.01 * v)
        else:
          v = jnp.maximum(v, 0.0)
        o_ref[pl.ds(i * 16, 16)] = v
        return c
      lax.fori_loop(0, Npad // 16, vact, 0)
      def vpool(i, c):
        lane = i * 16 + iota
        m = lane < Nn
        idx = jnp.where(m, PIDX[pl.ds(i * 16, 16)], 0)
        TMPC[pl.ds(i * 16, 16)] = plsc.load_gather(o_ref, [idx])
        return c
      lax.fori_loop(0, Nnp // 16, vpool, 0)
      pltpu.sync_copy(TMPC.at[pl.ds(0, Nnp)], SH_H.at[pl.ds(pl.multiple_of(row * NMAX, 8), Nnp)])

    pltpu.sync_copy(pool, PIDX.at[pl.ds(0, Nn)])
    bias_act_pool(O0, 0, wid)
    if Fo > 16:
      bias_act_pool(O1, 16, wid + 16)
    plsc.subcore_barrier()

  level(0, s0, d0, p1, WB1, BB1, SH_ES0, SH_ED0)
  level(1, s1, d1, p2, WB2, BB2, SH_ES1, SH_ED1)
  level(2, s2, d2, p3, WB3, BB3, SH_ES2, SH_ED2)
  level(3, s3, d3, p4, WB4, BB4, SH_ES3, SH_ED3)

  iot7 = jnp.minimum(iota, 7)
  msk8 = iota < 8
  acc = jnp.zeros((16,), F32)
  for q in range(2):
    cidx = 2 * wid + q
    pltpu.sync_copy(SH_H.at[pl.ds(pl.multiple_of(cidx * NMAX, 8), 32)], HC)
    def iloop(i, a, cidx=cidx):
      hs = plsc.load_gather(HC, [jnp.full((16,), i, I32)])
      r = jnp.full((16,), i * 32 * 8, I32) + cidx * 8
      wv = plsc.load_gather(FCWv, [r + iot7])
      wv = jnp.where(msk8, wv, 0.0)
      return a + hs * wv
    acc = lax.fori_loop(0, 22, iloop, acc)
  V16[...] = acc
  pltpu.sync_copy(V16, SH_RED.at[pl.ds(pl.multiple_of(wid * 16, 8), 16)])
  plsc.subcore_barrier()

  @pl.when(wid == 0)
  def _final():
    def ploop(p, a):
      pltpu.sync_copy(SH_RED.at[pl.ds(pl.multiple_of(p * 16, 8), 16)], V16)
      return a + V16[...]
    acc2 = lax.fori_loop(0, 16, ploop, jnp.zeros((16,), F32))
    fb = jnp.where(msk8, plsc.load_gather(FCBv, [iot7]), 0.0)
    V16[...] = acc2 + fb
    pltpu.sync_copy(V16.at[pl.ds(0, 8)], out)


@jax.jit
def _encoder_sc(xT, s0, d0, s1, d1, s2, d2, s3, d3,
                p1, p2, p3, p4, W1, b1, W2, b2, W3, b3, W4, b4, fcW, fcb):
  mesh = plsc.VectorSubcoreMesh(core_axis_name="c", subcore_axis_name="s",
                                num_cores=1)
  f = pl.kernel(
      _sc_body,
      out_type=jax.ShapeDtypeStruct((8,), F32),
      mesh=mesh,
      compiler_params=pltpu.CompilerParams(needs_layout_passes=False),
      scratch_types=[
          pltpu.VMEM((NMAX,), F32),
          pltpu.VMEM((NMAX,), F32),
          pltpu.VMEM((NMAX,), F32),
          pltpu.VMEM((NMAX,), F32),
          pltpu.VMEM((NMAX,), F32),
          pltpu.VMEM((NMAX,), F32),
          pltpu.VMEM((NMAX,), F32),
          pltpu.VMEM((CHMAX,), I32),
          pltpu.VMEM((CHMAX,), I32),
          pltpu.VMEM((CHMAX,), F32),
          pltpu.VMEM((1408,), I32),
          pltpu.VMEM((KCH * 3 * 16,), F32),
          pltpu.VMEM((KCH * 16 * 16,), F32),
          pltpu.VMEM((KCH * 16 * 16,), F32),
          pltpu.VMEM((KCH * 16 * 32,), F32),
          pltpu.VMEM((16,), F32),
          pltpu.VMEM((16,), F32),
          pltpu.VMEM((16,), F32),
          pltpu.VMEM((32,), F32),
          pltpu.VMEM((704 * 8,), F32),
          pltpu.VMEM((8,), F32),
          pltpu.VMEM((16,), F32),
          pltpu.VMEM((32,), F32),
          pltpu.VMEM_SHARED((32 * NMAX,), F32),
          pltpu.VMEM_SHARED((16 * NMAX,), F32),
          pltpu.VMEM_SHARED((16 * NMAX,), F32),
          pltpu.VMEM_SHARED((NMAX,), F32),
          pltpu.VMEM_SHARED((EMAX,), F32),
          pltpu.VMEM_SHARED((16 * 16,), F32),
          pltpu.VMEM_SHARED((33792,), I32),
          pltpu.VMEM_SHARED((33792,), I32),
          pltpu.VMEM_SHARED((8448,), I32),
          pltpu.VMEM_SHARED((8448,), I32),
          pltpu.VMEM_SHARED((2112,), I32),
          pltpu.VMEM_SHARED((2112,), I32),
          pltpu.VMEM_SHARED((528,), I32),
          pltpu.VMEM_SHARED((528,), I32),
      ],
  )
  return f(xT, s0, d0, s1, d1, s2, d2, s3, d3,
           p1, p2, p3, p4, W1.reshape(-1), b1, W2.reshape(-1), b2,
      W3.reshape(-1), b3, W4.reshape(-1), b4, fcW.reshape(-1), fcb)


def kernel(x, edge_index, edges1, edges2, edges3, pool1, pool2, pool3, pool4,
           W1, b1, W2, b2, W3, b3, W4, b4, fcW, fcb):
  xT = x.T.reshape(-1)
  return _encoder_sc(
      xT,
      edge_index[0], edge_index[1],
      edges1[0], edges1[1],
      edges2[0], edges2[1],
      edges3[0], edges3[1],
      pool1, pool2, pool3, pool4,
      W1, b1, W2, b2, W3, b3, W4, b4, fcW, fcb)


# speedup vs baseline: 6.9900x; 1.3291x over previous
import functools

import jax
import jax.numpy as jnp
from jax import lax
from jax.experimental import pallas as pl
from jax.experimental.pallas import tpu as pltpu
from jax.experimental.pallas import tpu_sc as plsc

F32 = jnp.float32
I32 = jnp.int32
KCH = 6

LVL = [
    dict(N=5632, Npad=5632, E=33792, CH=1024, Fi=3,  Fo=16, S=2112, R=352,
         Nn=1408, Nnp=1408, leaky=True),
    dict(N=1408, Npad=1536, E=8448,  CH=1056, Fi=16, Fo=16, S=528,  R=96,
         Nn=352, Nnp=352, leaky=False),
    dict(N=352,  Npad=512,  E=2112,  CH=704,  Fi=16, Fo=16, S=136,  R=32,
         Nn=88, Nnp=96, leaky=False),
    dict(N=88,   Npad=256,  E=528,   CH=528,  Fi=16, Fo=32, S=40,   R=16,
         Nn=22, Nnp=32, leaky=False),
]

NMAX = 5632
EMAX = 33792
CHMAX = 1056


def _splat(v, dt=I32):
  return jnp.full((16,), v, dt)


def _rsqrt_q(d):
  i = plsc.bitcast(d, I32)
  y = plsc.bitcast(0x5F3759DF - lax.shift_right_arithmetic(i, 1), F32)
  for _ in range(3):
    y = y * (1.5 - 0.5 * d * y * y)
  return y


def _zero(ref, n):
  def body(i, c):
    ref[pl.ds(i * 16, 16)] = jnp.zeros((16,), F32)
    return c
  lax.fori_loop(0, n // 16, body, 0)


def _sc_body(xT, s0, d0, s1, d1, s2, d2, s3, d3,
             p1, p2, p3, p4, W1, b1, W2, b2, W3, b3, W4, b4, fcW, fcb,
             out,
             A, B, C, O0, O1, TMPC, DIS, ES, ED, EN, PIDX, CES, CED, CEN,
             WB1, WB2, WB3, WB4, BB1, BB2, BB3, BB4, FCWv, FCBv, V16, HC,
             SH_H, SH_A, SH_P, SH_DIS, SH_EN, SH_RED,
             SH_ES0, SH_ED0, SH_ES1, SH_ED1, SH_ES2, SH_ED2, SH_ES3, SH_ED3):
  wid = lax.axis_index("s")
  iota = lax.iota(I32, 16)
  ones16 = jnp.ones((16,), F32)

  pltpu.sync_copy(W1, WB1)
  pltpu.sync_copy(W2, WB2)
  pltpu.sync_copy(W3, WB3)
  pltpu.sync_copy(W4, WB4)
  pltpu.sync_copy(b1, BB1)
  pltpu.sync_copy(b2, BB2)
  pltpu.sync_copy(b3, BB3)
  pltpu.sync_copy(b4, BB4)
  pltpu.sync_copy(fcW, FCWv)
  pltpu.sync_copy(fcb, FCBv)

  def matmul(k, WB, Fi, Fo, Npad):
    bank = (k % 2) * 16 * NMAX
    def jloop(j, c):
      pltpu.sync_copy(SH_A.at[pl.ds(pl.multiple_of(bank + j * NMAX, 8), Npad)], TMPC.at[pl.ds(0, Npad)])
      jbase = jnp.full((16,), k * Fi * Fo, I32) + j * Fo
      w0 = plsc.load_gather(WB, [jbase + wid])
      if Fo > 16:
        w1 = plsc.load_gather(WB, [jbase + wid + 16])
      def vloop(i, c2):
        t = TMPC[pl.ds(i * 16, 16)]
        O0[pl.ds(i * 16, 16)] += t * w0
        if Fo > 16:
          O1[pl.ds(i * 16, 16)] += t * w1
        return c2
      lax.fori_loop(0, Npad // 16, vloop, 0)
      return c
    lax.fori_loop(0, Fi, jloop, 0)

  def prop_cached(src_ref, dst_ref, nv, Npad, win=None):
    _zero(dst_ref, Npad)
    def vloop(j, c2):
      s = CES[pl.ds(j * 16, 16)]
      d = CED[pl.ds(j * 16, 16)]
      w = CEN[pl.ds(j * 16, 16)]
      val = plsc.load_gather(src_ref, [s]) * w
      if win is not None:
        base2, lo0, hi0 = win
        pos = base2 + j * 16 + iota
        val = jnp.where((pos >= lo0) & (pos < hi0), val, 0.0)
      plsc.addupdate_scatter(dst_ref, [d], val)
      return c2
    lax.fori_loop(0, nv, vloop, 0)

  def level(l, e_src, e_dst, pool, WB, BB, SH_ES, SH_ED):
    lv = LVL[l]
    N, Npad, E, CH = lv["N"], lv["Npad"], lv["E"], lv["CH"]
    Fi, Fo, S, R = lv["Fi"], lv["Fo"], lv["S"], lv["R"]
    Nn, Nnp, leaky = lv["Nn"], lv["Nnp"], lv["leaky"]
    nch_sl = -(-S // CH)

    @pl.when(wid == 0)
    def _stage():
      pltpu.sync_copy(e_src, SH_ES)
      pltpu.sync_copy(e_dst, SH_ED)
    plsc.subcore_barrier()

    _zero(C, Npad)
    lo = wid * S
    hi = jnp.minimum(lo + S, E)

    def degchunk(ci, c):
      base = lo + ci * CH
      base2 = pl.multiple_of(jnp.maximum(jnp.minimum(base, E - CH), 0), 8)
      pltpu.sync_copy(SH_ED.at[pl.ds(base2, CH)], ED.at[pl.ds(0, CH)])
      def vloop(j, c2):
        pos = base2 + j * 16 + iota
        m = (pos >= base) & (pos < hi)
        dd = ED[pl.ds(j * 16, 16)]
        plsc.addupdate_scatter(C, [dd], jnp.where(m, 1.0, 0.0).astype(F32))
        return c2
      lax.fori_loop(0, CH // 16, vloop, 0)
      return c
    lax.fori_loop(0, nch_sl, degchunk, 0)
    pltpu.sync_copy(C.at[pl.ds(0, Npad)], SH_P.at[pl.ds(pl.multiple_of(wid * NMAX, 8), Npad)])
    plsc.subcore_barrier()

    rbase = pl.multiple_of(wid * R, 8)
    _zero(A, R)
    def pacc(p, c):
      pltpu.sync_copy(SH_P.at[pl.ds(pl.multiple_of(p * NMAX + rbase, 8), R)], TMPC.at[pl.ds(0, R)])
      def vloop(j, c2):
        A[pl.ds(j * 16, 16)] += TMPC[pl.ds(j * 16, 16)]
        return c2
      lax.fori_loop(0, R // 16, vloop, 0)
      return c
    lax.fori_loop(0, 16, pacc, 0)
    def vdis(j, c):
      d = A[pl.ds(j * 16, 16)]
      B[pl.ds(j * 16, 16)] = jnp.where(d > 0.0, _rsqrt_q(d), 0.0)
      return c
    lax.fori_loop(0, R // 16, vdis, 0)
    pltpu.sync_copy(B.at[pl.ds(0, R)], SH_DIS.at[pl.ds(rbase, R)])
    plsc.subcore_barrier()

    pltpu.sync_copy(SH_DIS.at[pl.ds(0, Npad)], DIS.at[pl.ds(0, Npad)])
    def nchunk(ci, c):
      base = lo + ci * CH
      base2 = pl.multiple_of(jnp.maximum(jnp.minimum(base, E - CH), 0), 8)
      pltpu.sync_copy(SH_ES.at[pl.ds(base2, CH)], ES.at[pl.ds(0, CH)])
      pltpu.sync_copy(SH_ED.at[pl.ds(base2, CH)], ED.at[pl.ds(0, CH)])
      def vloop(j, c2):
        s = ES[pl.ds(j * 16, 16)]
        d = ED[pl.ds(j * 16, 16)]
        g = plsc.load_gather(DIS, [s]) * plsc.load_gather(DIS, [d])
        EN[pl.ds(j * 16, 16)] = -g
        return c2
      lax.fori_loop(0, CH // 16, vloop, 0)
      pltpu.sync_copy(EN.at[pl.ds(0, CH)], SH_EN.at[pl.ds(base2, CH)])
      return c
    lax.fori_loop(0, nch_sl, nchunk, 0)
    plsc.subcore_barrier()

    if l == 0:
      NSL, SL = 5, 6768
      NV0 = SL // 16
      col0 = wid // NSL
      sl = wid % NSL
      lo0 = sl * SL
      hi0 = jnp.minimum(lo0 + SL, E)
      cbase = pl.multiple_of(jnp.minimum(lo0, E - SL), 8)
      def fill0():
        pltpu.sync_copy(SH_ES.at[pl.ds(cbase, SL)], CES.at[pl.ds(0, SL)])
        pltpu.sync_copy(SH_ED.at[pl.ds(cbase, SL)], CED.at[pl.ds(0, SL)])
        pltpu.sync_copy(SH_EN.at[pl.ds(cbase, SL)], CEN.at[pl.ds(0, SL)])
      pl.when(wid < 15)(fill0)
      win = (cbase, lo0, hi0)
      arow = pl.multiple_of(col0 * NMAX, 8)
      prow = pl.multiple_of(wid * NMAX, 8)
      is_owner = (wid < 15) & (sl == 0)
      is_helper = (wid < 15) & (sl > 0)
    else:
      pltpu.sync_copy(SH_ES, CES.at[pl.ds(0, E)])
      pltpu.sync_copy(SH_ED, CED.at[pl.ds(0, E)])
      pltpu.sync_copy(SH_EN.at[pl.ds(0, E)], CEN.at[pl.ds(0, E)])
      arow = pl.multiple_of(wid * NMAX, 8)

    bufs = [A, B, C]

    def init_col(a_ref):
      if l == 0:
        pltpu.sync_copy(xT.at[pl.ds(arow, Npad)], a_ref.at[pl.ds(0, Npad)])
      else:
        pltpu.sync_copy(SH_H.at[pl.ds(arow, Npad)], a_ref.at[pl.ds(0, Npad)])
      pltpu.sync_copy(a_ref.at[pl.ds(0, Npad)], SH_A.at[pl.ds(arow, Npad)])

    def refresh(dst_ref, k):
      bnk = pl.multiple_of((k % 2) * 16 * NMAX + arow, 8)
      pltpu.sync_copy(SH_A.at[pl.ds(bnk, Npad)], dst_ref.at[pl.ds(0, Npad)])

    if l == 0:
      pl.when(is_owner)(functools.partial(init_col, bufs[0]))
    else:
      pl.when(wid < Fi)(functools.partial(init_col, bufs[0]))
    _zero(O0, Npad)
    if Fo > 16:
      _zero(O1, Npad)
    plsc.subcore_barrier()
    if l == 0:
      pl.when(is_helper)(functools.partial(refresh, bufs[0], 0))
    matmul(0, WB, Fi, Fo, Npad)

    for k in range(1, KCH):
      a, b, c = bufs
      srcb = a if k == 1 else b
      if l == 0:
        def scat0(srcb=srcb, c=c):
          prop_cached(srcb, c, NV0, Npad, win=win)
          pltpu.sync_copy(c.at[pl.ds(0, Npad)], SH_P.at[pl.ds(prow, Npad)])
        pl.when(wid < 15)(scat0)
        plsc.subcore_barrier()
        def red0(a=a, c=c, k=k):
          _zero(c, Npad)
          def pacc5(p, cc):
            pltpu.sync_copy(
                SH_P.at[pl.ds(pl.multiple_of((wid + p) * NMAX, 8), Npad)],
                TMPC.at[pl.ds(0, Npad)])
            def vadd(i, c2):
              c[pl.ds(i * 16, 16)] += TMPC[pl.ds(i * 16, 16)]
              return c2
            lax.fori_loop(0, Npad // 16, vadd, 0)
            return cc
          lax.fori_loop(0, NSL, pacc5, 0)
          if k > 1:
            def tr(i, cc):
              c[pl.ds(i * 16, 16)] = (2.0 * c[pl.ds(i * 16, 16)]
                                      - a[pl.ds(i * 16, 16)])
              return cc
            lax.fori_loop(0, Npad // 16, tr, 0)
          bnk = pl.multiple_of((k % 2) * 16 * NMAX + arow, 8)
          pltpu.sync_copy(c.at[pl.ds(0, Npad)], SH_A.at[pl.ds(bnk, Npad)])
        pl.when(is_owner)(red0)
      else:
        def stepk(srcb=srcb, a=a, c=c, k=k):
          prop_cached(srcb, c, E // 16, Npad)
          if k > 1:
            def tr(i, cc):
              c[pl.ds(i * 16, 16)] = (2.0 * c[pl.ds(i * 16, 16)]
                                      - a[pl.ds(i * 16, 16)])
              return cc
            lax.fori_loop(0, Npad // 16, tr, 0)
          bnk = pl.multiple_of((k % 2) * 16 * NMAX + arow, 8)
          pltpu.sync_copy(c.at[pl.ds(0, Npad)], SH_A.at[pl.ds(bnk, Npad)])
        pl.when(wid < Fi)(stepk)
      bufs = [a, c, b] if k == 1 else [b, c, a]
      plsc.subcore_barrier()
      if l == 0:
        pl.when(is_helper)(functools.partial(refresh, bufs[1], k))
      matmul(k, WB, Fi, Fo, Npad)

    def bias_act_pool(o_ref, col_off, row):
      bv = plsc.load_gather(BB, [_splat(col_off) + wid])
      def vact(i, c):
        v = o_ref[pl.ds(i * 16, 16)] + bv
        if leaky:
          v = jnp.maximum(v, 0.01 * v)
        else:
          v = jnp.maximum(v, 0.0)
        o_ref[pl.ds(i * 16, 16)] = v
        return c
      lax.fori_loop(0, Npad // 16, vact, 0)
      def vpool(i, c):
        lane = i * 16 + iota
        m = lane < Nn
        idx = jnp.where(m, PIDX[pl.ds(i * 16, 16)], 0)
        TMPC[pl.ds(i * 16, 16)] = plsc.load_gather(o_ref, [idx])
        return c
      lax.fori_loop(0, Nnp // 16, vpool, 0)
      pltpu.sync_copy(TMPC.at[pl.ds(0, Nnp)], SH_H.at[pl.ds(pl.multiple_of(row * NMAX, 8), Nnp)])

    pltpu.sync_copy(pool, PIDX.at[pl.ds(0, Nn)])
    bias_act_pool(O0, 0, wid)
    if Fo > 16:
      bias_act_pool(O1, 16, wid + 16)
    plsc.subcore_barrier()

  level(0, s0, d0, p1, WB1, BB1, SH_ES0, SH_ED0)
  level(1, s1, d1, p2, WB2, BB2, SH_ES1, SH_ED1)
  level(2, s2, d2, p3, WB3, BB3, SH_ES2, SH_ED2)
  level(3, s3, d3, p4, WB4, BB4, SH_ES3, SH_ED3)

  iot7 = jnp.minimum(iota, 7)
  msk8 = iota < 8
  acc = jnp.zeros((16,), F32)
  for q in range(2):
    cidx = 2 * wid + q
    pltpu.sync_copy(SH_H.at[pl.ds(pl.multiple_of(cidx * NMAX, 8), 32)], HC)
    def iloop(i, a, cidx=cidx):
      hs = plsc.load_gather(HC, [jnp.full((16,), i, I32)])
      r = jnp.full((16,), i * 32 * 8, I32) + cidx * 8
      wv = plsc.load_gather(FCWv, [r + iot7])
      wv = jnp.where(msk8, wv, 0.0)
      return a + hs * wv
    acc = lax.fori_loop(0, 22, iloop, acc)
  V16[...] = acc
  pltpu.sync_copy(V16, SH_RED.at[pl.ds(pl.multiple_of(wid * 16, 8), 16)])
  plsc.subcore_barrier()

  @pl.when(wid == 0)
  def _final():
    def ploop(p, a):
      pltpu.sync_copy(SH_RED.at[pl.ds(pl.multiple_of(p * 16, 8), 16)], V16)
      return a + V16[...]
    acc2 = lax.fori_loop(0, 16, ploop, jnp.zeros((16,), F32))
    fb = jnp.where(msk8, plsc.load_gather(FCBv, [iot7]), 0.0)
    V16[...] = acc2 + fb
    pltpu.sync_copy(V16.at[pl.ds(0, 8)], out)


@jax.jit
def _encoder_sc(xT, s0, d0, s1, d1, s2, d2, s3, d3,
                p1, p2, p3, p4, W1, b1, W2, b2, W3, b3, W4, b4, fcW, fcb):
  mesh = plsc.VectorSubcoreMesh(core_axis_name="c", subcore_axis_name="s",
                                num_cores=1)
  f = pl.kernel(
      _sc_body,
      out_type=jax.ShapeDtypeStruct((8,), F32),
      mesh=mesh,
      compiler_params=pltpu.CompilerParams(needs_layout_passes=False),
      scratch_types=[
          pltpu.VMEM((NMAX,), F32),
          pltpu.VMEM((NMAX,), F32),
          pltpu.VMEM((NMAX,), F32),
          pltpu.VMEM((NMAX,), F32),
          pltpu.VMEM((NMAX,), F32),
          pltpu.VMEM((NMAX,), F32),
          pltpu.VMEM((NMAX,), F32),
          pltpu.VMEM((CHMAX,), I32),
          pltpu.VMEM((CHMAX,), I32),
          pltpu.VMEM((CHMAX,), F32),
          pltpu.VMEM((1408,), I32),
          pltpu.VMEM((8448,), I32),
          pltpu.VMEM((8448,), I32),
          pltpu.VMEM((8448,), F32),
          pltpu.VMEM((KCH * 3 * 16,), F32),
          pltpu.VMEM((KCH * 16 * 16,), F32),
          pltpu.VMEM((KCH * 16 * 16,), F32),
          pltpu.VMEM((KCH * 16 * 32,), F32),
          pltpu.VMEM((16,), F32),
          pltpu.VMEM((16,), F32),
          pltpu.VMEM((16,), F32),
          pltpu.VMEM((32,), F32),
          pltpu.VMEM((704 * 8,), F32),
          pltpu.VMEM((8,), F32),
          pltpu.VMEM((16,), F32),
          pltpu.VMEM((32,), F32),
          pltpu.VMEM_SHARED((32 * NMAX,), F32),
          pltpu.VMEM_SHARED((32 * NMAX,), F32),
          pltpu.VMEM_SHARED((16 * NMAX,), F32),
          pltpu.VMEM_SHARED((NMAX,), F32),
          pltpu.VMEM_SHARED((EMAX,), F32),
          pltpu.VMEM_SHARED((16 * 16,), F32),
          pltpu.VMEM_SHARED((33792,), I32),
          pltpu.VMEM_SHARED((33792,), I32),
          pltpu.VMEM_SHARED((8448,), I32),
          pltpu.VMEM_SHARED((8448,), I32),
          pltpu.VMEM_SHARED((2112,), I32),
          pltpu.VMEM_SHARED((2112,), I32),
          pltpu.VMEM_SHARED((528,), I32),
          pltpu.VMEM_SHARED((528,), I32),
      ],
  )
  return f(xT, s0, d0, s1, d1, s2, d2, s3, d3,
           p1, p2, p3, p4, W1.reshape(-1), b1, W2.reshape(-1), b2,
      W3.reshape(-1), b3, W4.reshape(-1), b4, fcW.reshape(-1), fcb)


def kernel(x, edge_index, edges1, edges2, edges3, pool1, pool2, pool3, pool4,
           W1, b1, W2, b2, W3, b3, W4, b4, fcW, fcb):
  xT = x.T.reshape(-1)
  return _encoder_sc(
      xT,
      edge_index[0], edge_index[1],
      edges1[0], edges1[1],
      edges2[0], edges2[1],
      edges3[0], edges3[1],
      pool1, pool2, pool3, pool4,
      W1, b1, W2, b2, W3, b3, W4, b4, fcW, fcb)


# speedup vs baseline: 7.6223x; 1.0905x over previous
import functools

import jax
import jax.numpy as jnp
from jax import lax
from jax.experimental import pallas as pl
from jax.experimental.pallas import tpu as pltpu
from jax.experimental.pallas import tpu_sc as plsc

F32 = jnp.float32
I32 = jnp.int32
KCH = 6

LVL = [
    dict(N=5632, Npad=5632, E=33792, CH=1024, Fi=3,  Fo=16, S=2112, R=352,
         Nn=1408, Nnp=1408, leaky=True),
    dict(N=1408, Npad=1536, E=8448,  CH=1056, Fi=16, Fo=16, S=528,  R=96,
         Nn=352, Nnp=352, leaky=False),
    dict(N=352,  Npad=512,  E=2112,  CH=704,  Fi=16, Fo=16, S=136,  R=32,
         Nn=88, Nnp=96, leaky=False),
    dict(N=88,   Npad=256,  E=528,   CH=528,  Fi=16, Fo=32, S=40,   R=16,
         Nn=22, Nnp=32, leaky=False),
]

NMAX = 5632
EMAX = 33792
CHMAX = 1056


def _splat(v, dt=I32):
  return jnp.full((16,), v, dt)


def _rsqrt_q(d):
  i = plsc.bitcast(d, I32)
  y = plsc.bitcast(0x5F3759DF - lax.shift_right_arithmetic(i, 1), F32)
  for _ in range(3):
    y = y * (1.5 - 0.5 * d * y * y)
  return y


def _zero(ref, n):
  def body(i, c):
    ref[pl.ds(i * 16, 16)] = jnp.zeros((16,), F32)
    return c
  lax.fori_loop(0, n // 16, body, 0)


def _sc_body(xT, s0, d0, s1, d1, s2, d2, s3, d3,
             p1, p2, p3, p4, W1, b1, W2, b2, W3, b3, W4, b4, fcW, fcb,
             out,
             A, B, C, O0, O1, TMPC, DIS, ES, ED, EN, PIDX, CES, CED, CEN, TXB,
             WB1, WB2, WB3, WB4, BB1, BB2, BB3, BB4, FCWv, FCBv, V16, HC,
             SH_H, SH_A, SH_P, SH_DIS, SH_EN, SH_RED,
             SH_ES0, SH_ED0, SH_ES1, SH_ED1, SH_ES2, SH_ED2, SH_ES3, SH_ED3):
  wid = lax.axis_index("s")
  iota = lax.iota(I32, 16)
  ones16 = jnp.ones((16,), F32)

  pltpu.sync_copy(W1, WB1)
  pltpu.sync_copy(W2, WB2)
  pltpu.sync_copy(W3, WB3)
  pltpu.sync_copy(W4, WB4)
  pltpu.sync_copy(b1, BB1)
  pltpu.sync_copy(b2, BB2)
  pltpu.sync_copy(b3, BB3)
  pltpu.sync_copy(b4, BB4)
  pltpu.sync_copy(fcW, FCWv)
  pltpu.sync_copy(fcb, FCBv)

  def matmul(k, WB, Fi, Fo, Npad):
    bank = (k % 2) * Fi * Npad
    pltpu.sync_copy(SH_A.at[pl.ds(bank, Fi * Npad)], TXB.at[pl.ds(0, Fi * Npad)])
    def jloop(j, c):
      jv = jnp.full((16,), k * Fi * Fo, I32) + j * Fo
      w0 = plsc.load_gather(WB, [jv + wid])
      if Fo > 16:
        w1 = plsc.load_gather(WB, [jv + wid + 16])
      jbase = pl.multiple_of(j * Npad, 8)
      def vloop(i, c2):
        t = TXB[pl.ds(jbase + i * 16, 16)]
        O0[pl.ds(i * 16, 16)] += t * w0
        if Fo > 16:
          O1[pl.ds(i * 16, 16)] += t * w1
        return c2
      lax.fori_loop(0, Npad // 16, vloop, 0)
      return c
    lax.fori_loop(0, Fi, jloop, 0)

  def prop_cached(src_ref, dst_ref, nv, Npad, win=None):
    _zero(dst_ref, Npad)
    def vloop(j, c2):
      s = CES[pl.ds(j * 16, 16)]
      d = CED[pl.ds(j * 16, 16)]
      w = CEN[pl.ds(j * 16, 16)]
      val = plsc.load_gather(src_ref, [s]) * w
      if win is not None:
        base2, lo0, hi0 = win
        pos = base2 + j * 16 + iota
        val = jnp.where((pos >= lo0) & (pos < hi0), val, 0.0)
      plsc.addupdate_scatter(dst_ref, [d], val)
      return c2
    lax.fori_loop(0, nv, vloop, 0)

  def level(l, e_src, e_dst, pool, WB, BB, SH_ES, SH_ED):
    lv = LVL[l]
    N, Npad, E, CH = lv["N"], lv["Npad"], lv["E"], lv["CH"]
    Fi, Fo, S, R = lv["Fi"], lv["Fo"], lv["S"], lv["R"]
    Nn, Nnp, leaky = lv["Nn"], lv["Nnp"], lv["leaky"]
    nch_sl = -(-S // CH)

    @pl.when(wid == 0)
    def _stage():
      pltpu.sync_copy(e_src, SH_ES)
      pltpu.sync_copy(e_dst, SH_ED)
    plsc.subcore_barrier()

    _zero(C, Npad)
    lo = wid * S
    hi = jnp.minimum(lo + S, E)

    def degchunk(ci, c):
      base = lo + ci * CH
      base2 = pl.multiple_of(jnp.maximum(jnp.minimum(base, E - CH), 0), 8)
      pltpu.sync_copy(SH_ED.at[pl.ds(base2, CH)], ED.at[pl.ds(0, CH)])
      def vloop(j, c2):
        pos = base2 + j * 16 + iota
        m = (pos >= base) & (pos < hi)
        dd = ED[pl.ds(j * 16, 16)]
        plsc.addupdate_scatter(C, [dd], jnp.where(m, 1.0, 0.0).astype(F32))
        return c2
      lax.fori_loop(0, CH // 16, vloop, 0)
      return c
    lax.fori_loop(0, nch_sl, degchunk, 0)
    pltpu.sync_copy(C.at[pl.ds(0, Npad)], SH_P.at[pl.ds(pl.multiple_of(wid * NMAX, 8), Npad)])
    plsc.subcore_barrier()

    rbase = pl.multiple_of(wid * R, 8)
    _zero(A, R)
    def pacc(p, c):
      pltpu.sync_copy(SH_P.at[pl.ds(pl.multiple_of(p * NMAX + rbase, 8), R)], TMPC.at[pl.ds(0, R)])
      def vloop(j, c2):
        A[pl.ds(j * 16, 16)] += TMPC[pl.ds(j * 16, 16)]
        return c2
      lax.fori_loop(0, R // 16, vloop, 0)
      return c
    lax.fori_loop(0, 16, pacc, 0)
    def vdis(j, c):
      d = A[pl.ds(j * 16, 16)]
      B[pl.ds(j * 16, 16)] = jnp.where(d > 0.0, _rsqrt_q(d), 0.0)
      return c
    lax.fori_loop(0, R // 16, vdis, 0)
    pltpu.sync_copy(B.at[pl.ds(0, R)], SH_DIS.at[pl.ds(rbase, R)])
    plsc.subcore_barrier()

    pltpu.sync_copy(SH_DIS.at[pl.ds(0, Npad)], DIS.at[pl.ds(0, Npad)])
    def nchunk(ci, c):
      base = lo + ci * CH
      base2 = pl.multiple_of(jnp.maximum(jnp.minimum(base, E - CH), 0), 8)
      pltpu.sync_copy(SH_ES.at[pl.ds(base2, CH)], ES.at[pl.ds(0, CH)])
      pltpu.sync_copy(SH_ED.at[pl.ds(base2, CH)], ED.at[pl.ds(0, CH)])
      def vloop(j, c2):
        s = ES[pl.ds(j * 16, 16)]
        d = ED[pl.ds(j * 16, 16)]
        g = plsc.load_gather(DIS, [s]) * plsc.load_gather(DIS, [d])
        EN[pl.ds(j * 16, 16)] = -g
        return c2
      lax.fori_loop(0, CH // 16, vloop, 0)
      pltpu.sync_copy(EN.at[pl.ds(0, CH)], SH_EN.at[pl.ds(base2, CH)])
      return c
    lax.fori_loop(0, nch_sl, nchunk, 0)
    plsc.subcore_barrier()

    if l == 0:
      NSL, SL = 5, 6768
      NV0 = SL // 16
      col0 = wid // NSL
      sl = wid % NSL
      lo0 = sl * SL
      hi0 = jnp.minimum(lo0 + SL, E)
      cbase = pl.multiple_of(jnp.minimum(lo0, E - SL), 8)
      def fill0():
        pltpu.sync_copy(SH_ES.at[pl.ds(cbase, SL)], CES.at[pl.ds(0, SL)])
        pltpu.sync_copy(SH_ED.at[pl.ds(cbase, SL)], CED.at[pl.ds(0, SL)])
        pltpu.sync_copy(SH_EN.at[pl.ds(cbase, SL)], CEN.at[pl.ds(0, SL)])
      pl.when(wid < 15)(fill0)
      win = (cbase, lo0, hi0)
      arow = pl.multiple_of(col0 * NMAX, 8)
      apub = pl.multiple_of(col0 * Npad, 8)
      prow = pl.multiple_of(wid * NMAX, 8)
      is_owner = (wid < 15) & (sl == 0)
      is_helper = (wid < 15) & (sl > 0)
    else:
      pltpu.sync_copy(SH_ES, CES.at[pl.ds(0, E)])
      pltpu.sync_copy(SH_ED, CED.at[pl.ds(0, E)])
      pltpu.sync_copy(SH_EN.at[pl.ds(0, E)], CEN.at[pl.ds(0, E)])
      arow = pl.multiple_of(wid * NMAX, 8)
      apub = pl.multiple_of(wid * Npad, 8)

    bufs = [A, B, C]

    def init_col(a_ref):
      if l == 0:
        pltpu.sync_copy(xT.at[pl.ds(arow, Npad)], a_ref.at[pl.ds(0, Npad)])
      else:
        pltpu.sync_copy(SH_H.at[pl.ds(pl.multiple_of(wid * 1536, 8), Npad)], a_ref.at[pl.ds(0, Npad)])
      pltpu.sync_copy(a_ref.at[pl.ds(0, Npad)], SH_A.at[pl.ds(apub, Npad)])

    def refresh(dst_ref, k):
      bnk = pl.multiple_of((k % 2) * Fi * Npad + apub, 8)
      pltpu.sync_copy(SH_A.at[pl.ds(bnk, Npad)], dst_ref.at[pl.ds(0, Npad)])

    if l == 0:
      pl.when(is_owner)(functools.partial(init_col, bufs[0]))
    else:
      pl.when(wid < Fi)(functools.partial(init_col, bufs[0]))
    _zero(O0, Npad)
    if Fo > 16:
      _zero(O1, Npad)
    plsc.subcore_barrier()
    if l == 0:
      pl.when(is_helper)(functools.partial(refresh, bufs[0], 0))
    matmul(0, WB, Fi, Fo, Npad)

    for k in range(1, KCH):
      a, b, c = bufs
      srcb = a if k == 1 else b
      if l == 0:
        def scat0(srcb=srcb, c=c):
          prop_cached(srcb, c, NV0, Npad, win=win)
          pltpu.sync_copy(c.at[pl.ds(0, Npad)], SH_P.at[pl.ds(prow, Npad)])
        pl.when(wid < 15)(scat0)
        plsc.subcore_barrier()
        def red0(a=a, c=c, k=k):
          pltpu.sync_copy(SH_P.at[pl.ds(prow, NSL * NMAX)],
                          TXB.at[pl.ds(0, NSL * NMAX)])
          def vsum(i, cc):
            t = (TXB[pl.ds(i * 16, 16)]
                 + TXB[pl.ds(NMAX + i * 16, 16)]
                 + TXB[pl.ds(2 * NMAX + i * 16, 16)]
                 + TXB[pl.ds(3 * NMAX + i * 16, 16)]
                 + TXB[pl.ds(4 * NMAX + i * 16, 16)])
            c[pl.ds(i * 16, 16)] = t
            return cc
          lax.fori_loop(0, Npad // 16, vsum, 0)
          if k > 1:
            def tr(i, cc):
              c[pl.ds(i * 16, 16)] = (2.0 * c[pl.ds(i * 16, 16)]
                                      - a[pl.ds(i * 16, 16)])
              return cc
            lax.fori_loop(0, Npad // 16, tr, 0)
          bnk = pl.multiple_of((k % 2) * Fi * Npad + apub, 8)
          pltpu.sync_copy(c.at[pl.ds(0, Npad)], SH_A.at[pl.ds(bnk, Npad)])
        pl.when(is_owner)(red0)
      else:
        def stepk(srcb=srcb, a=a, c=c, k=k):
          prop_cached(srcb, c, E // 16, Npad)
          if k > 1:
            def tr(i, cc):
              c[pl.ds(i * 16, 16)] = (2.0 * c[pl.ds(i * 16, 16)]
                                      - a[pl.ds(i * 16, 16)])
              return cc
            lax.fori_loop(0, Npad // 16, tr, 0)
          bnk = pl.multiple_of((k % 2) * Fi * Npad + apub, 8)
          pltpu.sync_copy(c.at[pl.ds(0, Npad)], SH_A.at[pl.ds(bnk, Npad)])
        pl.when(wid < Fi)(stepk)
      bufs = [a, c, b] if k == 1 else [b, c, a]
      plsc.subcore_barrier()
      if l == 0:
        pl.when(is_helper)(functools.partial(refresh, bufs[1], k))
      matmul(k, WB, Fi, Fo, Npad)

    def bias_act_pool(o_ref, col_off, row):
      bv = plsc.load_gather(BB, [_splat(col_off) + wid])
      def vact(i, c):
        v = o_ref[pl.ds(i * 16, 16)] + bv
        if leaky:
          v = jnp.maximum(v, 0.01 * v)
        else:
          v = jnp.maximum(v, 0.0)
        o_ref[pl.ds(i * 16, 16)] = v
        return c
      lax.fori_loop(0, Npad // 16, vact, 0)
      def vpool(i, c):
        lane = i * 16 + iota
        m = lane < Nn
        idx = jnp.where(m, PIDX[pl.ds(i * 16, 16)], 0)
        TMPC[pl.ds(i * 16, 16)] = plsc.load_gather(o_ref, [idx])
        return c
      lax.fori_loop(0, Nnp // 16, vpool, 0)
      pltpu.sync_copy(TMPC.at[pl.ds(0, Nnp)], SH_H.at[pl.ds(pl.multiple_of(row * 1536, 8), Nnp)])

    pltpu.sync_copy(pool, PIDX.at[pl.ds(0, Nn)])
    bias_act_pool(O0, 0, wid)
    if Fo > 16:
      bias_act_pool(O1, 16, wid + 16)
    plsc.subcore_barrier()

  level(0, s0, d0, p1, WB1, BB1, SH_ES0, SH_ED0)
  level(1, s1, d1, p2, WB2, BB2, SH_ES1, SH_ED1)
  level(2, s2, d2, p3, WB3, BB3, SH_ES2, SH_ED2)
  level(3, s3, d3, p4, WB4, BB4, SH_ES3, SH_ED3)

  iot7 = jnp.minimum(iota, 7)
  msk8 = iota < 8
  acc = jnp.zeros((16,), F32)
  for q in range(2):
    cidx = 2 * wid + q
    pltpu.sync_copy(SH_H.at[pl.ds(pl.multiple_of(cidx * 1536, 8), 32)], HC)
    def iloop(i, a, cidx=cidx):
      hs = plsc.load_gather(HC, [jnp.full((16,), i, I32)])
      r = jnp.full((16,), i * 32 * 8, I32) + cidx * 8
      wv = plsc.load_gather(FCWv, [r + iot7])
      wv = jnp.where(msk8, wv, 0.0)
      return a + hs * wv
    acc = lax.fori_loop(0, 22, iloop, acc)
  V16[...] = acc
  pltpu.sync_copy(V16, SH_RED.at[pl.ds(pl.multiple_of(wid * 16, 8), 16)])
  plsc.subcore_barrier()

  @pl.when(wid == 0)
  def _final():
    def ploop(p, a):
      pltpu.sync_copy(SH_RED.at[pl.ds(pl.multiple_of(p * 16, 8), 16)], V16)
      return a + V16[...]
    acc2 = lax.fori_loop(0, 16, ploop, jnp.zeros((16,), F32))
    fb = jnp.where(msk8, plsc.load_gather(FCBv, [iot7]), 0.0)
    V16[...] = acc2 + fb
    pltpu.sync_copy(V16.at[pl.ds(0, 8)], out)


@jax.jit
def _encoder_sc(xT, s0, d0, s1, d1, s2, d2, s3, d3,
                p1, p2, p3, p4, W1, b1, W2, b2, W3, b3, W4, b4, fcW, fcb):
  mesh = plsc.VectorSubcoreMesh(core_axis_name="c", subcore_axis_name="s",
                                num_cores=1)
  f = pl.kernel(
      _sc_body,
      out_type=jax.ShapeDtypeStruct((8,), F32),
      mesh=mesh,
      compiler_params=pltpu.CompilerParams(needs_layout_passes=False),
      scratch_types=[
          pltpu.VMEM((NMAX,), F32),
          pltpu.VMEM((NMAX,), F32),
          pltpu.VMEM((NMAX,), F32),
          pltpu.VMEM((NMAX,), F32),
          pltpu.VMEM((NMAX,), F32),
          pltpu.VMEM((NMAX,), F32),
          pltpu.VMEM((NMAX,), F32),
          pltpu.VMEM((CHMAX,), I32),
          pltpu.VMEM((CHMAX,), I32),
          pltpu.VMEM((CHMAX,), F32),
          pltpu.VMEM((1408,), I32),
          pltpu.VMEM((8448,), I32),
          pltpu.VMEM((8448,), I32),
          pltpu.VMEM((8448,), F32),
          pltpu.VMEM((28160,), F32),
          pltpu.VMEM((KCH * 3 * 16,), F32),
          pltpu.VMEM((KCH * 16 * 16,), F32),
          pltpu.VMEM((KCH * 16 * 16,), F32),
          pltpu.VMEM((KCH * 16 * 32,), F32),
          pltpu.VMEM((16,), F32),
          pltpu.VMEM((16,), F32),
          pltpu.VMEM((16,), F32),
          pltpu.VMEM((32,), F32),
          pltpu.VMEM((704 * 8,), F32),
          pltpu.VMEM((8,), F32),
          pltpu.VMEM((16,), F32),
          pltpu.VMEM((32,), F32),
          pltpu.VMEM_SHARED((32 * 1536,), F32),
          pltpu.VMEM_SHARED((2 * 16 * 1536,), F32),
          pltpu.VMEM_SHARED((16 * NMAX,), F32),
          pltpu.VMEM_SHARED((NMAX,), F32),
          pltpu.VMEM_SHARED((EMAX,), F32),
          pltpu.VMEM_SHARED((16 * 16,), F32),
          pltpu.VMEM_SHARED((33792,), I32),
          pltpu.VMEM_SHARED((33792,), I32),
          pltpu.VMEM_SHARED((8448,), I32),
          pltpu.VMEM_SHARED((8448,), I32),
          pltpu.VMEM_SHARED((2112,), I32),
          pltpu.VMEM_SHARED((2112,), I32),
          pltpu.VMEM_SHARED((528,), I32),
          pltpu.VMEM_SHARED((528,), I32),
      ],
  )
  return f(xT, s0, d0, s1, d1, s2, d2, s3, d3,
           p1, p2, p3, p4, W1.reshape(-1), b1, W2.reshape(-1), b2,
      W3.reshape(-1), b3, W4.reshape(-1), b4, fcW.reshape(-1), fcb)


def kernel(x, edge_index, edges1, edges2, edges3, pool1, pool2, pool3, pool4,
           W1, b1, W2, b2, W3, b3, W4, b4, fcW, fcb):
  xT = x.T.reshape(-1)
  return _encoder_sc(
      xT,
      edge_index[0], edge_index[1],
      edges1[0], edges1[1],
      edges2[0], edges2[1],
      edges3[0], edges3[1],
      pool1, pool2, pool3, pool4,
      W1, b1, W2, b2, W3, b3, W4, b4, fcW, fcb)


# speedup vs baseline: 9.2646x; 1.2155x over previous
import functools

import jax
import jax.numpy as jnp
from jax import lax
from jax.experimental import pallas as pl
from jax.experimental.pallas import tpu as pltpu
from jax.experimental.pallas import tpu_sc as plsc

F32 = jnp.float32
I32 = jnp.int32
KCH = 6

LVL = [
    dict(N=5632, Npad=5632, E=33792, CH=1024, Fi=3,  Fo=16, S=2112, R=352,
         Nn=1408, Nnp=1408, leaky=True),
    dict(N=1408, Npad=1536, E=8448,  CH=1056, Fi=16, Fo=16, S=528,  R=96,
         Nn=352, Nnp=352, leaky=False),
    dict(N=352,  Npad=512,  E=2112,  CH=704,  Fi=16, Fo=16, S=136,  R=32,
         Nn=88, Nnp=96, leaky=False),
    dict(N=88,   Npad=256,  E=528,   CH=528,  Fi=16, Fo=32, S=40,   R=16,
         Nn=22, Nnp=32, leaky=False),
]

NMAX = 5632
EMAX = 33792
CHMAX = 1056


def _splat(v, dt=I32):
  return jnp.full((16,), v, dt)


def _rsqrt_q(d):
  i = plsc.bitcast(d, I32)
  y = plsc.bitcast(0x5F3759DF - lax.shift_right_arithmetic(i, 1), F32)
  for _ in range(3):
    y = y * (1.5 - 0.5 * d * y * y)
  return y


def _zero(ref, n):
  def body(i, c):
    ref[pl.ds(i * 16, 16)] = jnp.zeros((16,), F32)
    return c
  lax.fori_loop(0, n // 16, body, 0)


def _sc_body(xT, s0, d0, s1, d1, s2, d2, s3, d3,
             p1, p2, p3, p4, W1, b1, W2, b2, W3, b3, W4, b4, fcW, fcb,
             out,
             A, B, C, O0, O1, TMPC, DIS, ES, ED, EN, PIDX, CES, CED, CEN, TXB,
             WB1, WB2, WB3, WB4, BB1, BB2, BB3, BB4, FCWv, FCBv, V16, HC,
             SH_H, SH_A, SH_P, SH_DIS, SH_EN, SH_RED,
             SH_ES0, SH_ED0, SH_ES1, SH_ED1, SH_ES2, SH_ED2, SH_ES3, SH_ED3):
  wid = lax.axis_index("s")
  iota = lax.iota(I32, 16)
  ones16 = jnp.ones((16,), F32)

  pltpu.sync_copy(W1, WB1)
  pltpu.sync_copy(W2, WB2)
  pltpu.sync_copy(W3, WB3)
  pltpu.sync_copy(W4, WB4)
  pltpu.sync_copy(b1, BB1)
  pltpu.sync_copy(b2, BB2)
  pltpu.sync_copy(b3, BB3)
  pltpu.sync_copy(b4, BB4)
  pltpu.sync_copy(fcW, FCWv)
  pltpu.sync_copy(fcb, FCBv)

  def matmul(k, WB, Fi, Fo, Npad):
    bank = (k % 2) * Fi * Npad
    pltpu.sync_copy(SH_A.at[pl.ds(bank, Fi * Npad)], TXB.at[pl.ds(0, Fi * Npad)])
    kbase = jnp.full((16,), k * Fi * Fo, I32) + wid
    for j0 in range(0, Fi, 4):
      jb = min(4, Fi - j0)
      w0s = [plsc.load_gather(WB, [kbase + (j0 + j) * Fo]) for j in range(jb)]
      if Fo > 16:
        w1s = [plsc.load_gather(WB, [kbase + ((j0 + j) * Fo + 16)])
               for j in range(jb)]
      def vloop(i, c2, j0=j0, jb=jb, w0s=w0s,
                w1s=(w1s if Fo > 16 else None)):
        acc0 = O0[pl.ds(i * 16, 16)]
        if Fo > 16:
          acc1 = O1[pl.ds(i * 16, 16)]
        for j in range(jb):
          t = TXB[pl.ds((j0 + j) * Npad + i * 16, 16)]
          acc0 = acc0 + t * w0s[j]
          if Fo > 16:
            acc1 = acc1 + t * w1s[j]
        O0[pl.ds(i * 16, 16)] = acc0
        if Fo > 16:
          O1[pl.ds(i * 16, 16)] = acc1
        return c2
      lax.fori_loop(0, Npad // 16, vloop, 0)

  def prop_cached(src_ref, dst_ref, nv, Npad):
    _zero(dst_ref, Npad)
    def vloop(j, c2):
      s = CES[pl.ds(j * 16, 16)]
      d = CED[pl.ds(j * 16, 16)]
      w = CEN[pl.ds(j * 16, 16)]
      val = plsc.load_gather(src_ref, [s]) * w
      plsc.addupdate_scatter(dst_ref, [d], val)
      return c2
    lax.fori_loop(0, nv, vloop, 0)

  def level(l, e_src, e_dst, pool, WB, BB, SH_ES, SH_ED):
    lv = LVL[l]
    N, Npad, E, CH = lv["N"], lv["Npad"], lv["E"], lv["CH"]
    Fi, Fo, S, R = lv["Fi"], lv["Fo"], lv["S"], lv["R"]
    Nn, Nnp, leaky = lv["Nn"], lv["Nnp"], lv["leaky"]
    nch_sl = -(-S // CH)

    @pl.when(wid == 0)
    def _stage():
      pltpu.sync_copy(e_src, SH_ES)
      pltpu.sync_copy(e_dst, SH_ED)
    plsc.subcore_barrier()

    _zero(C, Npad)
    lo = wid * S
    hi = jnp.minimum(lo + S, E)

    def degchunk(ci, c):
      base = lo + ci * CH
      base2 = pl.multiple_of(jnp.maximum(jnp.minimum(base, E - CH), 0), 8)
      pltpu.sync_copy(SH_ED.at[pl.ds(base2, CH)], ED.at[pl.ds(0, CH)])
      def vloop(j, c2):
        pos = base2 + j * 16 + iota
        m = (pos >= base) & (pos < hi)
        dd = ED[pl.ds(j * 16, 16)]
        plsc.addupdate_scatter(C, [dd], jnp.where(m, 1.0, 0.0).astype(F32))
        return c2
      lax.fori_loop(0, CH // 16, vloop, 0)
      return c
    lax.fori_loop(0, nch_sl, degchunk, 0)
    pltpu.sync_copy(C.at[pl.ds(0, Npad)], SH_P.at[pl.ds(pl.multiple_of(wid * NMAX, 8), Npad)])
    plsc.subcore_barrier()

    rbase = pl.multiple_of(wid * R, 8)
    _zero(A, R)
    def pacc(p, c):
      pltpu.sync_copy(SH_P.at[pl.ds(pl.multiple_of(p * NMAX + rbase, 8), R)], TMPC.at[pl.ds(0, R)])
      def vloop(j, c2):
        A[pl.ds(j * 16, 16)] += TMPC[pl.ds(j * 16, 16)]
        return c2
      lax.fori_loop(0, R // 16, vloop, 0)
      return c
    lax.fori_loop(0, 16, pacc, 0)
    def vdis(j, c):
      d = A[pl.ds(j * 16, 16)]
      B[pl.ds(j * 16, 16)] = jnp.where(d > 0.0, _rsqrt_q(d), 0.0)
      return c
    lax.fori_loop(0, R // 16, vdis, 0)
    pltpu.sync_copy(B.at[pl.ds(0, R)], SH_DIS.at[pl.ds(rbase, R)])
    plsc.subcore_barrier()

    pltpu.sync_copy(SH_DIS.at[pl.ds(0, Npad)], DIS.at[pl.ds(0, Npad)])
    def nchunk(ci, c):
      base = lo + ci * CH
      base2 = pl.multiple_of(jnp.maximum(jnp.minimum(base, E - CH), 0), 8)
      pltpu.sync_copy(SH_ES.at[pl.ds(base2, CH)], ES.at[pl.ds(0, CH)])
      pltpu.sync_copy(SH_ED.at[pl.ds(base2, CH)], ED.at[pl.ds(0, CH)])
      def vloop(j, c2):
        s = ES[pl.ds(j * 16, 16)]
        d = ED[pl.ds(j * 16, 16)]
        g = plsc.load_gather(DIS, [s]) * plsc.load_gather(DIS, [d])
        EN[pl.ds(j * 16, 16)] = -g
        return c2
      lax.fori_loop(0, CH // 16, vloop, 0)
      pltpu.sync_copy(EN.at[pl.ds(0, CH)], SH_EN.at[pl.ds(base2, CH)])
      return c
    lax.fori_loop(0, nch_sl, nchunk, 0)
    plsc.subcore_barrier()

    if l == 0:
      NSL, SL = 5, 6768
      NV0 = SL // 16
      col0 = wid // NSL
      sl = wid % NSL
      lo0 = sl * SL
      hi0 = jnp.minimum(lo0 + SL, E)
      cbase = pl.multiple_of(jnp.minimum(lo0, E - SL), 8)
      def fill0():
        pltpu.sync_copy(SH_ES.at[pl.ds(cbase, SL)], CES.at[pl.ds(0, SL)])
        pltpu.sync_copy(SH_ED.at[pl.ds(cbase, SL)], CED.at[pl.ds(0, SL)])
        pltpu.sync_copy(SH_EN.at[pl.ds(cbase, SL)], CEN.at[pl.ds(0, SL)])
        def vmask(j, c2):
          pos = cbase + j * 16 + iota
          CEN[pl.ds(j * 16, 16)] = jnp.where(
              (pos >= lo0) & (pos < hi0), CEN[pl.ds(j * 16, 16)], 0.0)
          return c2
        lax.fori_loop(0, NV0, vmask, 0)
      pl.when(wid < 15)(fill0)
      arow = pl.multiple_of(col0 * NMAX, 8)
      apub = pl.multiple_of(col0 * Npad, 8)
      prow = pl.multiple_of(wid * NMAX, 8)
      is_owner = (wid < 15) & (sl == 0)
      is_helper = (wid < 15) & (sl > 0)
    else:
      pltpu.sync_copy(SH_ES, CES.at[pl.ds(0, E)])
      pltpu.sync_copy(SH_ED, CED.at[pl.ds(0, E)])
      pltpu.sync_copy(SH_EN.at[pl.ds(0, E)], CEN.at[pl.ds(0, E)])
      arow = pl.multiple_of(wid * NMAX, 8)
      apub = pl.multiple_of(wid * Npad, 8)

    bufs = [A, B, C]

    def init_col(a_ref):
      if l == 0:
        pltpu.sync_copy(xT.at[pl.ds(arow, Npad)], a_ref.at[pl.ds(0, Npad)])
      else:
        pltpu.sync_copy(SH_H.at[pl.ds(pl.multiple_of(wid * 1536, 8), Npad)], a_ref.at[pl.ds(0, Npad)])
      pltpu.sync_copy(a_ref.at[pl.ds(0, Npad)], SH_A.at[pl.ds(apub, Npad)])

    def refresh(dst_ref, k):
      bnk = pl.multiple_of((k % 2) * Fi * Npad + apub, 8)
      pltpu.sync_copy(SH_A.at[pl.ds(bnk, Npad)], dst_ref.at[pl.ds(0, Npad)])

    if l == 0:
      pl.when(is_owner)(functools.partial(init_col, bufs[0]))
    else:
      pl.when(wid < Fi)(functools.partial(init_col, bufs[0]))
    _zero(O0, Npad)
    if Fo > 16:
      _zero(O1, Npad)
    plsc.subcore_barrier()
    if l == 0:
      pl.when(is_helper)(functools.partial(refresh, bufs[0], 0))
    matmul(0, WB, Fi, Fo, Npad)

    for k in range(1, KCH):
      a, b, c = bufs
      srcb = a if k == 1 else b
      if l == 0:
        def scat0(srcb=srcb, c=c):
          prop_cached(srcb, c, NV0, Npad)
          pltpu.sync_copy(c.at[pl.ds(0, Npad)], SH_P.at[pl.ds(prow, Npad)])
        pl.when(wid < 15)(scat0)
        plsc.subcore_barrier()
        def red0(a=a, c=c, k=k):
          pltpu.sync_copy(SH_P.at[pl.ds(prow, NMAX)], c.at[pl.ds(0, NMAX)])
          for p in range(1, NSL):
            pltpu.sync_copy(SH_P.at[pl.ds(pl.multiple_of(prow + p * NMAX, 8),
                                          NMAX)],
                            TMPC.at[pl.ds(0, NMAX)])
            def vsum(i, cc):
              c[pl.ds(i * 16, 16)] += TMPC[pl.ds(i * 16, 16)]
              return cc
            lax.fori_loop(0, Npad // 16, vsum, 0)
          if k > 1:
            def tr(i, cc):
              c[pl.ds(i * 16, 16)] = (2.0 * c[pl.ds(i * 16, 16)]
                                      - a[pl.ds(i * 16, 16)])
              return cc
            lax.fori_loop(0, Npad // 16, tr, 0)
          bnk = pl.multiple_of((k % 2) * Fi * Npad + apub, 8)
          pltpu.sync_copy(c.at[pl.ds(0, Npad)], SH_A.at[pl.ds(bnk, Npad)])
        pl.when(is_owner)(red0)
      else:
        def stepk(srcb=srcb, a=a, c=c, k=k):
          prop_cached(srcb, c, E // 16, Npad)
          if k > 1:
            def tr(i, cc):
              c[pl.ds(i * 16, 16)] = (2.0 * c[pl.ds(i * 16, 16)]
                                      - a[pl.ds(i * 16, 16)])
              return cc
            lax.fori_loop(0, Npad // 16, tr, 0)
          bnk = pl.multiple_of((k % 2) * Fi * Npad + apub, 8)
          pltpu.sync_copy(c.at[pl.ds(0, Npad)], SH_A.at[pl.ds(bnk, Npad)])
        pl.when(wid < Fi)(stepk)
      bufs = [a, c, b] if k == 1 else [b, c, a]
      plsc.subcore_barrier()
      if l == 0:
        pl.when(is_helper)(functools.partial(refresh, bufs[1], k))
      matmul(k, WB, Fi, Fo, Npad)

    def bias_act_pool(o_ref, col_off, row):
      bv = plsc.load_gather(BB, [_splat(col_off) + wid])
      def vact(i, c):
        v = o_ref[pl.ds(i * 16, 16)] + bv
        if leaky:
          v = jnp.maximum(v, 0.01 * v)
        else:
          v = jnp.maximum(v, 0.0)
        o_ref[pl.ds(i * 16, 16)] = v
        return c
      lax.fori_loop(0, Npad // 16, vact, 0)
      def vpool(i, c):
        lane = i * 16 + iota
        m = lane < Nn
        idx = jnp.where(m, PIDX[pl.ds(i * 16, 16)], 0)
        TMPC[pl.ds(i * 16, 16)] = plsc.load_gather(o_ref, [idx])
        return c
      lax.fori_loop(0, Nnp // 16, vpool, 0)
      pltpu.sync_copy(TMPC.at[pl.ds(0, Nnp)], SH_H.at[pl.ds(pl.multiple_of(row * 1536, 8), Nnp)])

    pltpu.sync_copy(pool, PIDX.at[pl.ds(0, Nn)])
    bias_act_pool(O0, 0, wid)
    if Fo > 16:
      bias_act_pool(O1, 16, wid + 16)
    plsc.subcore_barrier()

  level(0, s0, d0, p1, WB1, BB1, SH_ES0, SH_ED0)
  level(1, s1, d1, p2, WB2, BB2, SH_ES1, SH_ED1)
  level(2, s2, d2, p3, WB3, BB3, SH_ES2, SH_ED2)
  level(3, s3, d3, p4, WB4, BB4, SH_ES3, SH_ED3)

  iot7 = jnp.minimum(iota, 7)
  msk8 = iota < 8
  acc = jnp.zeros((16,), F32)
  for q in range(2):
    cidx = 2 * wid + q
    pltpu.sync_copy(SH_H.at[pl.ds(pl.multiple_of(cidx * 1536, 8), 32)], HC)
    def iloop(i, a, cidx=cidx):
      hs = plsc.load_gather(HC, [jnp.full((16,), i, I32)])
      r = jnp.full((16,), i * 32 * 8, I32) + cidx * 8
      wv = plsc.load_gather(FCWv, [r + iot7])
      wv = jnp.where(msk8, wv, 0.0)
      return a + hs * wv
    acc = lax.fori_loop(0, 22, iloop, acc)
  V16[...] = acc
  pltpu.sync_copy(V16, SH_RED.at[pl.ds(pl.multiple_of(wid * 16, 8), 16)])
  plsc.subcore_barrier()

  @pl.when(wid == 0)
  def _final():
    def ploop(p, a):
      pltpu.sync_copy(SH_RED.at[pl.ds(pl.multiple_of(p * 16, 8), 16)], V16)
      return a + V16[...]
    acc2 = lax.fori_loop(0, 16, ploop, jnp.zeros((16,), F32))
    fb = jnp.where(msk8, plsc.load_gather(FCBv, [iot7]), 0.0)
    V16[...] = acc2 + fb
    pltpu.sync_copy(V16.at[pl.ds(0, 8)], out)


@jax.jit
def _encoder_sc(xT, s0, d0, s1, d1, s2, d2, s3, d3,
                p1, p2, p3, p4, W1, b1, W2, b2, W3, b3, W4, b4, fcW, fcb):
  mesh = plsc.VectorSubcoreMesh(core_axis_name="c", subcore_axis_name="s",
                                num_cores=1)
  f = pl.kernel(
      _sc_body,
      out_type=jax.ShapeDtypeStruct((8,), F32),
      mesh=mesh,
      compiler_params=pltpu.CompilerParams(needs_layout_passes=False),
      scratch_types=[
          pltpu.VMEM((NMAX,), F32),
          pltpu.VMEM((NMAX,), F32),
          pltpu.VMEM((NMAX,), F32),
          pltpu.VMEM((NMAX,), F32),
          pltpu.VMEM((NMAX,), F32),
          pltpu.VMEM((NMAX,), F32),
          pltpu.VMEM((NMAX,), F32),
          pltpu.VMEM((CHMAX,), I32),
          pltpu.VMEM((CHMAX,), I32),
          pltpu.VMEM((CHMAX,), F32),
          pltpu.VMEM((1408,), I32),
          pltpu.VMEM((8448,), I32),
          pltpu.VMEM((8448,), I32),
          pltpu.VMEM((8448,), F32),
          pltpu.VMEM((24576,), F32),
          pltpu.VMEM((KCH * 3 * 16,), F32),
          pltpu.VMEM((KCH * 16 * 16,), F32),
          pltpu.VMEM((KCH * 16 * 16,), F32),
          pltpu.VMEM((KCH * 16 * 32,), F32),
          pltpu.VMEM((16,), F32),
          pltpu.VMEM((16,), F32),
          pltpu.VMEM((16,), F32),
          pltpu.VMEM((32,), F32),
          pltpu.VMEM((704 * 8,), F32),
          pltpu.VMEM((8,), F32),
          pltpu.VMEM((16,), F32),
          pltpu.VMEM((32,), F32),
          pltpu.VMEM_SHARED((32 * 1536,), F32),
          pltpu.VMEM_SHARED((2 * 16 * 1536,), F32),
          pltpu.VMEM_SHARED((16 * NMAX,), F32),
          pltpu.VMEM_SHARED((NMAX,), F32),
          pltpu.VMEM_SHARED((EMAX,), F32),
          pltpu.VMEM_SHARED((16 * 16,), F32),
          pltpu.VMEM_SHARED((33792,), I32),
          pltpu.VMEM_SHARED((33792,), I32),
          pltpu.VMEM_SHARED((8448,), I32),
          pltpu.VMEM_SHARED((8448,), I32),
          pltpu.VMEM_SHARED((2112,), I32),
          pltpu.VMEM_SHARED((2112,), I32),
          pltpu.VMEM_SHARED((528,), I32),
          pltpu.VMEM_SHARED((528,), I32),
      ],
  )
  return f(xT, s0, d0, s1, d1, s2, d2, s3, d3,
           p1, p2, p3, p4, W1.reshape(-1), b1, W2.reshape(-1), b2,
      W3.reshape(-1), b3, W4.reshape(-1), b4, fcW.reshape(-1), fcb)


def kernel(x, edge_index, edges1, edges2, edges3, pool1, pool2, pool3, pool4,
           W1, b1, W2, b2, W3, b3, W4, b4, fcW, fcb):
  xT = x.T.reshape(-1)
  return _encoder_sc(
      xT,
      edge_index[0], edge_index[1],
      edges1[0], edges1[1],
      edges2[0], edges2[1],
      edges3[0], edges3[1],
      pool1, pool2, pool3, pool4,
      W1, b1, W2, b2, W3, b3, W4, b4, fcW, fcb)


# speedup vs baseline: 13.3731x; 1.4435x over previous
import functools

import jax
import jax.numpy as jnp
from jax import lax
from jax.experimental import pallas as pl
from jax.experimental.pallas import tpu as pltpu
from jax.experimental.pallas import tpu_sc as plsc

F32 = jnp.float32
I32 = jnp.int32
KCH = 6

LVL = [
    dict(N=5632, Npad=5632, E=33792, CH=1024, Fi=3,  Fo=16, S=2112, R=352,
         Nn=1408, Nnp=1408, leaky=True),
    dict(N=1408, Npad=1536, E=8448,  CH=1056, Fi=16, Fo=16, S=528,  R=96,
         Nn=352, Nnp=352, leaky=False),
    dict(N=352,  Npad=512,  E=2112,  CH=704,  Fi=16, Fo=16, S=136,  R=32,
         Nn=88, Nnp=96, leaky=False),
    dict(N=88,   Npad=256,  E=528,   CH=528,  Fi=16, Fo=32, S=40,   R=16,
         Nn=22, Nnp=32, leaky=False),
]

NMAX = 5632
EMAX = 33792
CHMAX = 1056


def _splat(v, dt=I32):
  return jnp.full((16,), v, dt)


def _rsqrt_q(d):
  i = plsc.bitcast(d, I32)
  y = plsc.bitcast(0x5F3759DF - lax.shift_right_arithmetic(i, 1), F32)
  for _ in range(3):
    y = y * (1.5 - 0.5 * d * y * y)
  return y


def _zero(ref, n):
  @plsc.parallel_loop(0, n // 16, unroll=4)
  def body(i):
    ref[pl.ds(i * 16, 16)] = jnp.zeros((16,), F32)


def _sc_body(xT, s0, d0, s1, d1, s2, d2, s3, d3,
             p1, p2, p3, p4, W1, b1, W2, b2, W3, b3, W4, b4, fcW, fcb,
             out,
             A, B, C, O0, O1, TMPC, DIS, ES, ED, EN, PIDX, CES, CED, CEN, TXB,
             WB1, WB2, WB3, WB4, BB1, BB2, BB3, BB4, FCWv, FCBv, V16, HC,
             SH_H, SH_A, SH_P, SH_DIS, SH_EN, SH_RED,
             SH_ES0, SH_ED0, SH_ES1, SH_ED1, SH_ES2, SH_ED2, SH_ES3, SH_ED3):
  wid = lax.axis_index("s")
  iota = lax.iota(I32, 16)
  ones16 = jnp.ones((16,), F32)

  pltpu.sync_copy(W1, WB1)
  pltpu.sync_copy(W2, WB2)
  pltpu.sync_copy(W3, WB3)
  pltpu.sync_copy(W4, WB4)
  pltpu.sync_copy(b1, BB1)
  pltpu.sync_copy(b2, BB2)
  pltpu.sync_copy(b3, BB3)
  pltpu.sync_copy(b4, BB4)
  pltpu.sync_copy(fcW, FCWv)
  pltpu.sync_copy(fcb, FCBv)

  def matmul(k, WB, Fi, Fo, Npad):
    bank = (k % 2) * Fi * Npad
    pltpu.sync_copy(SH_A.at[pl.ds(bank, Fi * Npad)], TXB.at[pl.ds(0, Fi * Npad)])
    kbase = jnp.full((16,), k * Fi * Fo, I32) + wid
    for j0 in range(0, Fi, 8):
      jb = min(8, Fi - j0)
      w0s = [plsc.load_gather(WB, [kbase + (j0 + j) * Fo]) for j in range(jb)]
      if Fo > 16:
        w1s = [plsc.load_gather(WB, [kbase + ((j0 + j) * Fo + 16)])
               for j in range(jb)]
      @plsc.parallel_loop(0, Npad // 16, unroll=2)
      def vloop(i, j0=j0, jb=jb, w0s=w0s, w1s=(w1s if Fo > 16 else None)):
        acc0 = O0[pl.ds(i * 16, 16)]
        if Fo > 16:
          acc1 = O1[pl.ds(i * 16, 16)]
        for j in range(jb):
          t = TXB[pl.ds((j0 + j) * Npad + i * 16, 16)]
          acc0 = acc0 + t * w0s[j]
          if Fo > 16:
            acc1 = acc1 + t * w1s[j]
        O0[pl.ds(i * 16, 16)] = acc0
        if Fo > 16:
          O1[pl.ds(i * 16, 16)] = acc1

  def prop_cached(src_ref, dst_ref, nv, Npad):
    _zero(dst_ref, Npad)
    def vloop(j, c2):
      s = CES[pl.ds(j * 16, 16)]
      d = CED[pl.ds(j * 16, 16)]
      w = CEN[pl.ds(j * 16, 16)]
      val = plsc.load_gather(src_ref, [s]) * w
      plsc.addupdate_scatter(dst_ref, [d], val)
      return c2
    lax.fori_loop(0, nv, vloop, 0)

  def level(l, e_src, e_dst, pool, WB, BB, SH_ES, SH_ED):
    lv = LVL[l]
    N, Npad, E, CH = lv["N"], lv["Npad"], lv["E"], lv["CH"]
    Fi, Fo, S, R = lv["Fi"], lv["Fo"], lv["S"], lv["R"]
    Nn, Nnp, leaky = lv["Nn"], lv["Nnp"], lv["leaky"]
    nch_sl = -(-S // CH)

    @pl.when(wid == 0)
    def _stage():
      pltpu.sync_copy(e_src, SH_ES)
      pltpu.sync_copy(e_dst, SH_ED)
    plsc.subcore_barrier()

    _zero(C, Npad)
    lo = wid * S
    hi = jnp.minimum(lo + S, E)

    def degchunk(ci, c):
      base = lo + ci * CH
      base2 = pl.multiple_of(jnp.maximum(jnp.minimum(base, E - CH), 0), 8)
      pltpu.sync_copy(SH_ED.at[pl.ds(base2, CH)], ED.at[pl.ds(0, CH)])
      def vloop(j, c2):
        pos = base2 + j * 16 + iota
        m = (pos >= base) & (pos < hi)
        dd = ED[pl.ds(j * 16, 16)]
        plsc.addupdate_scatter(C, [dd], jnp.where(m, 1.0, 0.0).astype(F32))
        return c2
      lax.fori_loop(0, CH // 16, vloop, 0)
      return c
    lax.fori_loop(0, nch_sl, degchunk, 0)
    pltpu.sync_copy(C.at[pl.ds(0, Npad)], SH_P.at[pl.ds(pl.multiple_of(wid * NMAX, 8), Npad)])
    plsc.subcore_barrier()

    rbase = pl.multiple_of(wid * R, 8)
    _zero(A, R)
    def pacc(p, c):
      pltpu.sync_copy(SH_P.at[pl.ds(pl.multiple_of(p * NMAX + rbase, 8), R)], TMPC.at[pl.ds(0, R)])
      @plsc.parallel_loop(0, R // 16, unroll=4)
      def vloop(j):
        A[pl.ds(j * 16, 16)] += TMPC[pl.ds(j * 16, 16)]
      return c
    lax.fori_loop(0, 16, pacc, 0)
    @plsc.parallel_loop(0, R // 16, unroll=2)
    def vdis(j):
      d = A[pl.ds(j * 16, 16)]
      B[pl.ds(j * 16, 16)] = jnp.where(d > 0.0, _rsqrt_q(d), 0.0)
    pltpu.sync_copy(B.at[pl.ds(0, R)], SH_DIS.at[pl.ds(rbase, R)])
    plsc.subcore_barrier()

    pltpu.sync_copy(SH_DIS.at[pl.ds(0, Npad)], DIS.at[pl.ds(0, Npad)])
    def nchunk(ci, c):
      base = lo + ci * CH
      base2 = pl.multiple_of(jnp.maximum(jnp.minimum(base, E - CH), 0), 8)
      pltpu.sync_copy(SH_ES.at[pl.ds(base2, CH)], ES.at[pl.ds(0, CH)])
      pltpu.sync_copy(SH_ED.at[pl.ds(base2, CH)], ED.at[pl.ds(0, CH)])
      @plsc.parallel_loop(0, CH // 16, unroll=4)
      def vloop(j):
        s = ES[pl.ds(j * 16, 16)]
        d = ED[pl.ds(j * 16, 16)]
        g = plsc.load_gather(DIS, [s]) * plsc.load_gather(DIS, [d])
        EN[pl.ds(j * 16, 16)] = -g
      pltpu.sync_copy(EN.at[pl.ds(0, CH)], SH_EN.at[pl.ds(base2, CH)])
      return c
    lax.fori_loop(0, nch_sl, nchunk, 0)
    plsc.subcore_barrier()

    if l == 0:
      NSL, SL = 5, 6768
      NV0 = SL // 16
      col0 = wid // NSL
      sl = wid % NSL
      lo0 = sl * SL
      hi0 = jnp.minimum(lo0 + SL, E)
      cbase = pl.multiple_of(jnp.minimum(lo0, E - SL), 8)
      def fill0():
        pltpu.sync_copy(SH_ES.at[pl.ds(cbase, SL)], CES.at[pl.ds(0, SL)])
        pltpu.sync_copy(SH_ED.at[pl.ds(cbase, SL)], CED.at[pl.ds(0, SL)])
        pltpu.sync_copy(SH_EN.at[pl.ds(cbase, SL)], CEN.at[pl.ds(0, SL)])
        def vmask(j, c2):
          pos = cbase + j * 16 + iota
          CEN[pl.ds(j * 16, 16)] = jnp.where(
              (pos >= lo0) & (pos < hi0), CEN[pl.ds(j * 16, 16)], 0.0)
          return c2
        lax.fori_loop(0, NV0, vmask, 0)
      pl.when(wid < 15)(fill0)
      arow = pl.multiple_of(col0 * NMAX, 8)
      apub = pl.multiple_of(col0 * Npad, 8)
      prow = pl.multiple_of(wid * NMAX, 8)
      is_owner = (wid < 15) & (sl == 0)
      is_helper = (wid < 15) & (sl > 0)
    else:
      pltpu.sync_copy(SH_ES, CES.at[pl.ds(0, E)])
      pltpu.sync_copy(SH_ED, CED.at[pl.ds(0, E)])
      pltpu.sync_copy(SH_EN.at[pl.ds(0, E)], CEN.at[pl.ds(0, E)])
      arow = pl.multiple_of(wid * NMAX, 8)
      apub = pl.multiple_of(wid * Npad, 8)

    bufs = [A, B, C]

    def init_col(a_ref):
      if l == 0:
        pltpu.sync_copy(xT.at[pl.ds(arow, Npad)], a_ref.at[pl.ds(0, Npad)])
      else:
        pltpu.sync_copy(SH_H.at[pl.ds(pl.multiple_of(wid * 1536, 8), Npad)], a_ref.at[pl.ds(0, Npad)])
      pltpu.sync_copy(a_ref.at[pl.ds(0, Npad)], SH_A.at[pl.ds(apub, Npad)])

    def refresh(dst_ref, k):
      bnk = pl.multiple_of((k % 2) * Fi * Npad + apub, 8)
      pltpu.sync_copy(SH_A.at[pl.ds(bnk, Npad)], dst_ref.at[pl.ds(0, Npad)])

    if l == 0:
      pl.when(is_owner)(functools.partial(init_col, bufs[0]))
    else:
      pl.when(wid < Fi)(functools.partial(init_col, bufs[0]))
    _zero(O0, Npad)
    if Fo > 16:
      _zero(O1, Npad)
    plsc.subcore_barrier()
    if l == 0:
      pl.when(is_helper)(functools.partial(refresh, bufs[0], 0))
    matmul(0, WB, Fi, Fo, Npad)

    for k in range(1, KCH):
      a, b, c = bufs
      srcb = a if k == 1 else b
      if l == 0:
        def scat0(srcb=srcb, c=c):
          prop_cached(srcb, c, NV0, Npad)
          pltpu.sync_copy(c.at[pl.ds(0, Npad)], SH_P.at[pl.ds(prow, Npad)])
        pl.when(wid < 15)(scat0)
        plsc.subcore_barrier()
        def red0(a=a, c=c, k=k):
          pltpu.sync_copy(SH_P.at[pl.ds(prow, NMAX)], c.at[pl.ds(0, NMAX)])
          for p in range(1, NSL):
            pltpu.sync_copy(SH_P.at[pl.ds(pl.multiple_of(prow + p * NMAX, 8),
                                          NMAX)],
                            TMPC.at[pl.ds(0, NMAX)])
            @plsc.parallel_loop(0, Npad // 16, unroll=4)
            def vsum(i, c=c):
              c[pl.ds(i * 16, 16)] += TMPC[pl.ds(i * 16, 16)]
          if k > 1:
            @plsc.parallel_loop(0, Npad // 16, unroll=4)
            def tr(i, a=a, c=c):
              c[pl.ds(i * 16, 16)] = (2.0 * c[pl.ds(i * 16, 16)]
                                      - a[pl.ds(i * 16, 16)])
          bnk = pl.multiple_of((k % 2) * Fi * Npad + apub, 8)
          pltpu.sync_copy(c.at[pl.ds(0, Npad)], SH_A.at[pl.ds(bnk, Npad)])
        pl.when(is_owner)(red0)
      else:
        def stepk(srcb=srcb, a=a, c=c, k=k):
          prop_cached(srcb, c, E // 16, Npad)
          if k > 1:
            @plsc.parallel_loop(0, Npad // 16, unroll=4)
            def tr(i, a=a, c=c):
              c[pl.ds(i * 16, 16)] = (2.0 * c[pl.ds(i * 16, 16)]
                                      - a[pl.ds(i * 16, 16)])
          bnk = pl.multiple_of((k % 2) * Fi * Npad + apub, 8)
          pltpu.sync_copy(c.at[pl.ds(0, Npad)], SH_A.at[pl.ds(bnk, Npad)])
        pl.when(wid < Fi)(stepk)
      bufs = [a, c, b] if k == 1 else [b, c, a]
      plsc.subcore_barrier()
      if l == 0:
        pl.when(is_helper)(functools.partial(refresh, bufs[1], k))
      matmul(k, WB, Fi, Fo, Npad)

    def bias_act_pool(o_ref, col_off, row):
      bv = plsc.load_gather(BB, [_splat(col_off) + wid])
      @plsc.parallel_loop(0, Npad // 16, unroll=4)
      def vact(i):
        v = o_ref[pl.ds(i * 16, 16)] + bv
        if leaky:
          v = jnp.maximum(v, 0.01 * v)
        else:
          v = jnp.maximum(v, 0.0)
        o_ref[pl.ds(i * 16, 16)] = v
      @plsc.parallel_loop(0, Nnp // 16, unroll=2)
      def vpool(i):
        lane = i * 16 + iota
        m = lane < Nn
        idx = jnp.where(m, PIDX[pl.ds(i * 16, 16)], 0)
        TMPC[pl.ds(i * 16, 16)] = plsc.load_gather(o_ref, [idx])
      pltpu.sync_copy(TMPC.at[pl.ds(0, Nnp)], SH_H.at[pl.ds(pl.multiple_of(row * 1536, 8), Nnp)])

    pltpu.sync_copy(pool, PIDX.at[pl.ds(0, Nn)])
    bias_act_pool(O0, 0, wid)
    if Fo > 16:
      bias_act_pool(O1, 16, wid + 16)
    plsc.subcore_barrier()

  level(0, s0, d0, p1, WB1, BB1, SH_ES0, SH_ED0)
  level(1, s1, d1, p2, WB2, BB2, SH_ES1, SH_ED1)
  level(2, s2, d2, p3, WB3, BB3, SH_ES2, SH_ED2)
  level(3, s3, d3, p4, WB4, BB4, SH_ES3, SH_ED3)

  iot7 = jnp.minimum(iota, 7)
  msk8 = iota < 8
  acc = jnp.zeros((16,), F32)
  for q in range(2):
    cidx = 2 * wid + q
    pltpu.sync_copy(SH_H.at[pl.ds(pl.multiple_of(cidx * 1536, 8), 32)], HC)
    def iloop(i, a, cidx=cidx):
      hs = plsc.load_gather(HC, [jnp.full((16,), i, I32)])
      r = jnp.full((16,), i * 32 * 8, I32) + cidx * 8
      wv = plsc.load_gather(FCWv, [r + iot7])
      wv = jnp.where(msk8, wv, 0.0)
      return a + hs * wv
    acc = lax.fori_loop(0, 22, iloop, acc)
  V16[...] = acc
  pltpu.sync_copy(V16, SH_RED.at[pl.ds(pl.multiple_of(wid * 16, 8), 16)])
  plsc.subcore_barrier()

  @pl.when(wid == 0)
  def _final():
    def ploop(p, a):
      pltpu.sync_copy(SH_RED.at[pl.ds(pl.multiple_of(p * 16, 8), 16)], V16)
      return a + V16[...]
    acc2 = lax.fori_loop(0, 16, ploop, jnp.zeros((16,), F32))
    fb = jnp.where(msk8, plsc.load_gather(FCBv, [iot7]), 0.0)
    V16[...] = acc2 + fb
    pltpu.sync_copy(V16.at[pl.ds(0, 8)], out)


@jax.jit
def _encoder_sc(xT, s0, d0, s1, d1, s2, d2, s3, d3,
                p1, p2, p3, p4, W1, b1, W2, b2, W3, b3, W4, b4, fcW, fcb):
  mesh = plsc.VectorSubcoreMesh(core_axis_name="c", subcore_axis_name="s",
                                num_cores=1)
  f = pl.kernel(
      _sc_body,
      out_type=jax.ShapeDtypeStruct((8,), F32),
      mesh=mesh,
      compiler_params=pltpu.CompilerParams(needs_layout_passes=False),
      scratch_types=[
          pltpu.VMEM((NMAX,), F32),
          pltpu.VMEM((NMAX,), F32),
          pltpu.VMEM((NMAX,), F32),
          pltpu.VMEM((NMAX,), F32),
          pltpu.VMEM((NMAX,), F32),
          pltpu.VMEM((NMAX,), F32),
          pltpu.VMEM((NMAX,), F32),
          pltpu.VMEM((CHMAX,), I32),
          pltpu.VMEM((CHMAX,), I32),
          pltpu.VMEM((CHMAX,), F32),
          pltpu.VMEM((1408,), I32),
          pltpu.VMEM((8448,), I32),
          pltpu.VMEM((8448,), I32),
          pltpu.VMEM((8448,), F32),
          pltpu.VMEM((24576,), F32),
          pltpu.VMEM((KCH * 3 * 16,), F32),
          pltpu.VMEM((KCH * 16 * 16,), F32),
          pltpu.VMEM((KCH * 16 * 16,), F32),
          pltpu.VMEM((KCH * 16 * 32,), F32),
          pltpu.VMEM((16,), F32),
          pltpu.VMEM((16,), F32),
          pltpu.VMEM((16,), F32),
          pltpu.VMEM((32,), F32),
          pltpu.VMEM((704 * 8,), F32),
          pltpu.VMEM((8,), F32),
          pltpu.VMEM((16,), F32),
          pltpu.VMEM((32,), F32),
          pltpu.VMEM_SHARED((32 * 1536,), F32),
          pltpu.VMEM_SHARED((2 * 16 * 1536,), F32),
          pltpu.VMEM_SHARED((16 * NMAX,), F32),
          pltpu.VMEM_SHARED((NMAX,), F32),
          pltpu.VMEM_SHARED((EMAX,), F32),
          pltpu.VMEM_SHARED((16 * 16,), F32),
          pltpu.VMEM_SHARED((33792,), I32),
          pltpu.VMEM_SHARED((33792,), I32),
          pltpu.VMEM_SHARED((8448,), I32),
          pltpu.VMEM_SHARED((8448,), I32),
          pltpu.VMEM_SHARED((2112,), I32),
          pltpu.VMEM_SHARED((2112,), I32),
          pltpu.VMEM_SHARED((528,), I32),
          pltpu.VMEM_SHARED((528,), I32),
      ],
  )
  return f(xT, s0, d0, s1, d1, s2, d2, s3, d3,
           p1, p2, p3, p4, W1.reshape(-1), b1, W2.reshape(-1), b2,
      W3.reshape(-1), b3, W4.reshape(-1), b4, fcW.reshape(-1), fcb)


def kernel(x, edge_index, edges1, edges2, edges3, pool1, pool2, pool3, pool4,
           W1, b1, W2, b2, W3, b3, W4, b4, fcW, fcb):
  xT = x.T.reshape(-1)
  return _encoder_sc(
      xT,
      edge_index[0], edge_index[1],
      edges1[0], edges1[1],
      edges2[0], edges2[1],
      edges3[0], edges3[1],
      pool1, pool2, pool3, pool4,
      W1, b1, W2, b2, W3, b3, W4, b4, fcW, fcb)


# speedup vs baseline: 16.9002x; 1.2637x over previous
import functools

import jax
import jax.numpy as jnp
from jax import lax
from jax.experimental import pallas as pl
from jax.experimental.pallas import tpu as pltpu
from jax.experimental.pallas import tpu_sc as plsc

F32 = jnp.float32
I32 = jnp.int32
KCH = 6

LVL = [
    dict(N=5632, Npad=5632, E=33792, CH=1024, Fi=3,  Fo=16, S=2112, R=352,
         Nn=1408, Nnp=1408, leaky=True),
    dict(N=1408, Npad=1536, E=8448,  CH=1056, Fi=16, Fo=16, S=528,  R=96,
         Nn=352, Nnp=352, leaky=False),
    dict(N=352,  Npad=512,  E=2112,  CH=704,  Fi=16, Fo=16, S=136,  R=32,
         Nn=88, Nnp=96, leaky=False),
    dict(N=88,   Npad=256,  E=528,   CH=528,  Fi=16, Fo=32, S=40,   R=16,
         Nn=22, Nnp=32, leaky=False),
]

NMAX = 5632
EMAX = 33792
CHMAX = 1056


def _splat(v, dt=I32):
  return jnp.full((16,), v, dt)


def _rsqrt_q(d):
  i = plsc.bitcast(d, I32)
  y = plsc.bitcast(0x5F3759DF - lax.shift_right_arithmetic(i, 1), F32)
  for _ in range(3):
    y = y * (1.5 - 0.5 * d * y * y)
  return y


def _zero(ref, n):
  @plsc.parallel_loop(0, n // 16, unroll=4)
  def body(i):
    ref[pl.ds(i * 16, 16)] = jnp.zeros((16,), F32)


def _sc_body(xT, s0, d0, s1, d1, s2, d2, s3, d3,
             p1, p2, p3, p4, W1, b1, W2, b2, W3, b3, W4, b4, fcW, fcb,
             out,
             A, B, C, O0, O1, TMPC, DIS, ES, ED, EN, PIDX, CES, CED, CEN, TXB,
             WB1, WB2, WB3, WB4, BB1, BB2, BB3, BB4, FCWv, FCBv, V16, HC,
             SH_H, SH_A, SH_P, SH_DIS, SH_EN, SH_RED,
             SH_ES0, SH_ED0, SH_ES1, SH_ED1, SH_ES2, SH_ED2, SH_ES3, SH_ED3):
  wid = lax.axis_index("s")
  iota = lax.iota(I32, 16)
  ones16 = jnp.ones((16,), F32)

  pltpu.sync_copy(W1, WB1)
  pltpu.sync_copy(W2, WB2)
  pltpu.sync_copy(W3, WB3)
  pltpu.sync_copy(W4, WB4)
  pltpu.sync_copy(b1, BB1)
  pltpu.sync_copy(b2, BB2)
  pltpu.sync_copy(b3, BB3)
  pltpu.sync_copy(b4, BB4)
  pltpu.sync_copy(fcW, FCWv)
  pltpu.sync_copy(fcb, FCBv)

  def matmul(k, WB, Fi, Fo, Npad):
    bank = (k % 2) * Fi * Npad
    pltpu.sync_copy(SH_A.at[pl.ds(bank, Fi * Npad)], TXB.at[pl.ds(0, Fi * Npad)])
    kbase = jnp.full((16,), k * Fi * Fo, I32) + wid
    for j0 in range(0, Fi, 8):
      jb = min(8, Fi - j0)
      w0s = [plsc.load_gather(WB, [kbase + (j0 + j) * Fo]) for j in range(jb)]
      if Fo > 16:
        w1s = [plsc.load_gather(WB, [kbase + ((j0 + j) * Fo + 16)])
               for j in range(jb)]
      @plsc.parallel_loop(0, Npad // 16, unroll=2)
      def vloop(i, j0=j0, jb=jb, w0s=w0s, w1s=(w1s if Fo > 16 else None)):
        acc0 = O0[pl.ds(i * 16, 16)]
        if Fo > 16:
          acc1 = O1[pl.ds(i * 16, 16)]
        for j in range(jb):
          t = TXB[pl.ds((j0 + j) * Npad + i * 16, 16)]
          acc0 = acc0 + t * w0s[j]
          if Fo > 16:
            acc1 = acc1 + t * w1s[j]
        O0[pl.ds(i * 16, 16)] = acc0
        if Fo > 16:
          O1[pl.ds(i * 16, 16)] = acc1

  def prop_cached(src_ref, dst_ref, nv, Npad):
    _zero(dst_ref, Npad)
    @plsc.parallel_loop(0, nv, unroll=4)
    def vloop(j):
      s = CES[pl.ds(j * 16, 16)]
      d = CED[pl.ds(j * 16, 16)]
      w = CEN[pl.ds(j * 16, 16)]
      val = plsc.load_gather(src_ref, [s]) * w
      plsc.addupdate_scatter(dst_ref, [d], val)

  def level(l, e_src, e_dst, pool, WB, BB, SH_ES, SH_ED):
    lv = LVL[l]
    N, Npad, E, CH = lv["N"], lv["Npad"], lv["E"], lv["CH"]
    Fi, Fo, S, R = lv["Fi"], lv["Fo"], lv["S"], lv["R"]
    Nn, Nnp, leaky = lv["Nn"], lv["Nnp"], lv["leaky"]
    nch_sl = -(-S // CH)

    @pl.when(wid == 0)
    def _stage():
      pltpu.sync_copy(e_src, SH_ES)
      pltpu.sync_copy(e_dst, SH_ED)
    plsc.subcore_barrier()

    _zero(C, Npad)
    lo = wid * S
    hi = jnp.minimum(lo + S, E)

    def degchunk(ci, c):
      base = lo + ci * CH
      base2 = pl.multiple_of(jnp.maximum(jnp.minimum(base, E - CH), 0), 8)
      pltpu.sync_copy(SH_ED.at[pl.ds(base2, CH)], ED.at[pl.ds(0, CH)])
      @plsc.parallel_loop(0, CH // 16, unroll=4)
      def vloop(j):
        pos = base2 + j * 16 + iota
        m = (pos >= base) & (pos < hi)
        dd = ED[pl.ds(j * 16, 16)]
        plsc.addupdate_scatter(C, [dd], jnp.where(m, 1.0, 0.0).astype(F32))
      return c
    lax.fori_loop(0, nch_sl, degchunk, 0)
    pltpu.sync_copy(C.at[pl.ds(0, Npad)], SH_P.at[pl.ds(pl.multiple_of(wid * NMAX, 8), Npad)])
    plsc.subcore_barrier()

    rbase = pl.multiple_of(wid * R, 8)
    _zero(A, R)
    def pacc(p, c):
      pltpu.sync_copy(SH_P.at[pl.ds(pl.multiple_of(p * NMAX + rbase, 8), R)], TMPC.at[pl.ds(0, R)])
      @plsc.parallel_loop(0, R // 16, unroll=4)
      def vloop(j):
        A[pl.ds(j * 16, 16)] += TMPC[pl.ds(j * 16, 16)]
      return c
    lax.fori_loop(0, 16, pacc, 0)
    @plsc.parallel_loop(0, R // 16, unroll=2)
    def vdis(j):
      d = A[pl.ds(j * 16, 16)]
      B[pl.ds(j * 16, 16)] = jnp.where(d > 0.0, _rsqrt_q(d), 0.0)
    pltpu.sync_copy(B.at[pl.ds(0, R)], SH_DIS.at[pl.ds(rbase, R)])
    plsc.subcore_barrier()

    pltpu.sync_copy(SH_DIS.at[pl.ds(0, Npad)], DIS.at[pl.ds(0, Npad)])
    def nchunk(ci, c):
      base = lo + ci * CH
      base2 = pl.multiple_of(jnp.maximum(jnp.minimum(base, E - CH), 0), 8)
      pltpu.sync_copy(SH_ES.at[pl.ds(base2, CH)], ES.at[pl.ds(0, CH)])
      pltpu.sync_copy(SH_ED.at[pl.ds(base2, CH)], ED.at[pl.ds(0, CH)])
      @plsc.parallel_loop(0, CH // 16, unroll=4)
      def vloop(j):
        s = ES[pl.ds(j * 16, 16)]
        d = ED[pl.ds(j * 16, 16)]
        g = plsc.load_gather(DIS, [s]) * plsc.load_gather(DIS, [d])
        EN[pl.ds(j * 16, 16)] = -g
      pltpu.sync_copy(EN.at[pl.ds(0, CH)], SH_EN.at[pl.ds(base2, CH)])
      return c
    lax.fori_loop(0, nch_sl, nchunk, 0)
    plsc.subcore_barrier()

    if l == 0:
      NSL, SL = 5, 6768
      NV0 = SL // 16
      col0 = wid // NSL
      sl = wid % NSL
      lo0 = sl * SL
      hi0 = jnp.minimum(lo0 + SL, E)
      cbase = pl.multiple_of(jnp.minimum(lo0, E - SL), 8)
      def fill0():
        pltpu.sync_copy(SH_ES.at[pl.ds(cbase, SL)], CES.at[pl.ds(0, SL)])
        pltpu.sync_copy(SH_ED.at[pl.ds(cbase, SL)], CED.at[pl.ds(0, SL)])
        pltpu.sync_copy(SH_EN.at[pl.ds(cbase, SL)], CEN.at[pl.ds(0, SL)])
        def vmask(j, c2):
          pos = cbase + j * 16 + iota
          CEN[pl.ds(j * 16, 16)] = jnp.where(
              (pos >= lo0) & (pos < hi0), CEN[pl.ds(j * 16, 16)], 0.0)
          return c2
        lax.fori_loop(0, NV0, vmask, 0)
      pl.when(wid < 15)(fill0)
      arow = pl.multiple_of(col0 * NMAX, 8)
      apub = pl.multiple_of(col0 * Npad, 8)
      prow = pl.multiple_of(wid * NMAX, 8)
      is_owner = (wid < 15) & (sl == 0)
      is_helper = (wid < 15) & (sl > 0)
    else:
      pltpu.sync_copy(SH_ES, CES.at[pl.ds(0, E)])
      pltpu.sync_copy(SH_ED, CED.at[pl.ds(0, E)])
      pltpu.sync_copy(SH_EN.at[pl.ds(0, E)], CEN.at[pl.ds(0, E)])
      arow = pl.multiple_of(wid * NMAX, 8)
      apub = pl.multiple_of(wid * Npad, 8)

    bufs = [A, B, C]

    def init_col(a_ref):
      if l == 0:
        pltpu.sync_copy(xT.at[pl.ds(arow, Npad)], a_ref.at[pl.ds(0, Npad)])
      else:
        pltpu.sync_copy(SH_H.at[pl.ds(pl.multiple_of(wid * 1536, 8), Npad)], a_ref.at[pl.ds(0, Npad)])
      pltpu.sync_copy(a_ref.at[pl.ds(0, Npad)], SH_A.at[pl.ds(apub, Npad)])

    def refresh(dst_ref, k):
      bnk = pl.multiple_of((k % 2) * Fi * Npad + apub, 8)
      pltpu.sync_copy(SH_A.at[pl.ds(bnk, Npad)], dst_ref.at[pl.ds(0, Npad)])

    if l == 0:
      pl.when(is_owner)(functools.partial(init_col, bufs[0]))
    else:
      pl.when(wid < Fi)(functools.partial(init_col, bufs[0]))
    _zero(O0, Npad)
    if Fo > 16:
      _zero(O1, Npad)
    plsc.subcore_barrier()
    if l == 0:
      pl.when(is_helper)(functools.partial(refresh, bufs[0], 0))
    matmul(0, WB, Fi, Fo, Npad)

    for k in range(1, KCH):
      a, b, c = bufs
      srcb = a if k == 1 else b
      if l == 0:
        def scat0(srcb=srcb, c=c):
          prop_cached(srcb, c, NV0, Npad)
          pltpu.sync_copy(c.at[pl.ds(0, Npad)], SH_P.at[pl.ds(prow, Npad)])
        pl.when(wid < 15)(scat0)
        plsc.subcore_barrier()
        def red0(a=a, c=c, k=k):
          pltpu.sync_copy(SH_P.at[pl.ds(prow, NMAX)], c.at[pl.ds(0, NMAX)])
          for p in range(1, NSL):
            pltpu.sync_copy(SH_P.at[pl.ds(pl.multiple_of(prow + p * NMAX, 8),
                                          NMAX)],
                            TMPC.at[pl.ds(0, NMAX)])
            @plsc.parallel_loop(0, Npad // 16, unroll=4)
            def vsum(i, c=c):
              c[pl.ds(i * 16, 16)] += TMPC[pl.ds(i * 16, 16)]
          if k > 1:
            @plsc.parallel_loop(0, Npad // 16, unroll=4)
            def tr(i, a=a, c=c):
              c[pl.ds(i * 16, 16)] = (2.0 * c[pl.ds(i * 16, 16)]
                                      - a[pl.ds(i * 16, 16)])
          bnk = pl.multiple_of((k % 2) * Fi * Npad + apub, 8)
          pltpu.sync_copy(c.at[pl.ds(0, Npad)], SH_A.at[pl.ds(bnk, Npad)])
        pl.when(is_owner)(red0)
      else:
        def stepk(srcb=srcb, a=a, c=c, k=k):
          prop_cached(srcb, c, E // 16, Npad)
          if k > 1:
            @plsc.parallel_loop(0, Npad // 16, unroll=4)
            def tr(i, a=a, c=c):
              c[pl.ds(i * 16, 16)] = (2.0 * c[pl.ds(i * 16, 16)]
                                      - a[pl.ds(i * 16, 16)])
          bnk = pl.multiple_of((k % 2) * Fi * Npad + apub, 8)
          pltpu.sync_copy(c.at[pl.ds(0, Npad)], SH_A.at[pl.ds(bnk, Npad)])
        pl.when(wid < Fi)(stepk)
      bufs = [a, c, b] if k == 1 else [b, c, a]
      plsc.subcore_barrier()
      if l == 0:
        pl.when(is_helper)(functools.partial(refresh, bufs[1], k))
      matmul(k, WB, Fi, Fo, Npad)

    def bias_act_pool(o_ref, col_off, row):
      bv = plsc.load_gather(BB, [_splat(col_off) + wid])
      @plsc.parallel_loop(0, Npad // 16, unroll=4)
      def vact(i):
        v = o_ref[pl.ds(i * 16, 16)] + bv
        if leaky:
          v = jnp.maximum(v, 0.01 * v)
        else:
          v = jnp.maximum(v, 0.0)
        o_ref[pl.ds(i * 16, 16)] = v
      @plsc.parallel_loop(0, Nnp // 16, unroll=2)
      def vpool(i):
        lane = i * 16 + iota
        m = lane < Nn
        idx = jnp.where(m, PIDX[pl.ds(i * 16, 16)], 0)
        TMPC[pl.ds(i * 16, 16)] = plsc.load_gather(o_ref, [idx])
      pltpu.sync_copy(TMPC.at[pl.ds(0, Nnp)], SH_H.at[pl.ds(pl.multiple_of(row * 1536, 8), Nnp)])

    pltpu.sync_copy(pool, PIDX.at[pl.ds(0, Nn)])
    bias_act_pool(O0, 0, wid)
    if Fo > 16:
      bias_act_pool(O1, 16, wid + 16)
    plsc.subcore_barrier()

  level(0, s0, d0, p1, WB1, BB1, SH_ES0, SH_ED0)
  level(1, s1, d1, p2, WB2, BB2, SH_ES1, SH_ED1)
  level(2, s2, d2, p3, WB3, BB3, SH_ES2, SH_ED2)
  level(3, s3, d3, p4, WB4, BB4, SH_ES3, SH_ED3)

  iot7 = jnp.minimum(iota, 7)
  msk8 = iota < 8
  acc = jnp.zeros((16,), F32)
  for q in range(2):
    cidx = 2 * wid + q
    pltpu.sync_copy(SH_H.at[pl.ds(pl.multiple_of(cidx * 1536, 8), 32)], HC)
    def iloop(i, a, cidx=cidx):
      hs = plsc.load_gather(HC, [jnp.full((16,), i, I32)])
      r = jnp.full((16,), i * 32 * 8, I32) + cidx * 8
      wv = plsc.load_gather(FCWv, [r + iot7])
      wv = jnp.where(msk8, wv, 0.0)
      return a + hs * wv
    acc = lax.fori_loop(0, 22, iloop, acc)
  V16[...] = acc
  pltpu.sync_copy(V16, SH_RED.at[pl.ds(pl.multiple_of(wid * 16, 8), 16)])
  plsc.subcore_barrier()

  @pl.when(wid == 0)
  def _final():
    def ploop(p, a):
      pltpu.sync_copy(SH_RED.at[pl.ds(pl.multiple_of(p * 16, 8), 16)], V16)
      return a + V16[...]
    acc2 = lax.fori_loop(0, 16, ploop, jnp.zeros((16,), F32))
    fb = jnp.where(msk8, plsc.load_gather(FCBv, [iot7]), 0.0)
    V16[...] = acc2 + fb
    pltpu.sync_copy(V16.at[pl.ds(0, 8)], out)


@jax.jit
def _encoder_sc(xT, s0, d0, s1, d1, s2, d2, s3, d3,
                p1, p2, p3, p4, W1, b1, W2, b2, W3, b3, W4, b4, fcW, fcb):
  mesh = plsc.VectorSubcoreMesh(core_axis_name="c", subcore_axis_name="s",
                                num_cores=1)
  f = pl.kernel(
      _sc_body,
      out_type=jax.ShapeDtypeStruct((8,), F32),
      mesh=mesh,
      compiler_params=pltpu.CompilerParams(needs_layout_passes=False),
      scratch_types=[
          pltpu.VMEM((NMAX,), F32),
          pltpu.VMEM((NMAX,), F32),
          pltpu.VMEM((NMAX,), F32),
          pltpu.VMEM((NMAX,), F32),
          pltpu.VMEM((NMAX,), F32),
          pltpu.VMEM((NMAX,), F32),
          pltpu.VMEM((NMAX,), F32),
          pltpu.VMEM((CHMAX,), I32),
          pltpu.VMEM((CHMAX,), I32),
          pltpu.VMEM((CHMAX,), F32),
          pltpu.VMEM((1408,), I32),
          pltpu.VMEM((8448,), I32),
          pltpu.VMEM((8448,), I32),
          pltpu.VMEM((8448,), F32),
          pltpu.VMEM((24576,), F32),
          pltpu.VMEM((KCH * 3 * 16,), F32),
          pltpu.VMEM((KCH * 16 * 16,), F32),
          pltpu.VMEM((KCH * 16 * 16,), F32),
          pltpu.VMEM((KCH * 16 * 32,), F32),
          pltpu.VMEM((16,), F32),
          pltpu.VMEM((16,), F32),
          pltpu.VMEM((16,), F32),
          pltpu.VMEM((32,), F32),
          pltpu.VMEM((704 * 8,), F32),
          pltpu.VMEM((8,), F32),
          pltpu.VMEM((16,), F32),
          pltpu.VMEM((32,), F32),
          pltpu.VMEM_SHARED((32 * 1536,), F32),
          pltpu.VMEM_SHARED((2 * 16 * 1536,), F32),
          pltpu.VMEM_SHARED((16 * NMAX,), F32),
          pltpu.VMEM_SHARED((NMAX,), F32),
          pltpu.VMEM_SHARED((EMAX,), F32),
          pltpu.VMEM_SHARED((16 * 16,), F32),
          pltpu.VMEM_SHARED((33792,), I32),
          pltpu.VMEM_SHARED((33792,), I32),
          pltpu.VMEM_SHARED((8448,), I32),
          pltpu.VMEM_SHARED((8448,), I32),
          pltpu.VMEM_SHARED((2112,), I32),
          pltpu.VMEM_SHARED((2112,), I32),
          pltpu.VMEM_SHARED((528,), I32),
          pltpu.VMEM_SHARED((528,), I32),
      ],
  )
  return f(xT, s0, d0, s1, d1, s2, d2, s3, d3,
           p1, p2, p3, p4, W1.reshape(-1), b1, W2.reshape(-1), b2,
      W3.reshape(-1), b3, W4.reshape(-1), b4, fcW.reshape(-1), fcb)


def kernel(x, edge_index, edges1, edges2, edges3, pool1, pool2, pool3, pool4,
           W1, b1, W2, b2, W3, b3, W4, b4, fcW, fcb):
  xT = x.T.reshape(-1)
  return _encoder_sc(
      xT,
      edge_index[0], edge_index[1],
      edges1[0], edges1[1],
      edges2[0], edges2[1],
      edges3[0], edges3[1],
      pool1, pool2, pool3, pool4,
      W1, b1, W2, b2, W3, b3, W4, b4, fcW, fcb)
